# Initial kernel scaffold; baseline (speedup 1.0000x reference)
import jax
import jax.numpy as jnp
from jax.experimental import pallas as pl


def kernel(edge_index, edge_feat, node_feat, graph_attr, params):
    raise NotImplementedError("write your pallas kernel here")


# speedup vs baseline: 2.0132x; 2.0132x over previous
import functools

import jax
import jax.numpy as jnp
from jax import lax
from jax.experimental import pallas as pl
from jax.experimental.pallas import tpu as pltpu
from jax.experimental.pallas import tpu_sc as plsc

F32 = jnp.float32
_G = 128
_NW = 32


def _sp(x):
    return jnp.maximum(x, 0.0) + jnp.log(1.0 + jnp.exp(-jnp.abs(x)))


def _sig(x):
    return 1.0 / (1.0 + jnp.exp(-x))


def _dot(a, b):
    return lax.dot_general(a, b, (((1,), (0,)), ((), ())),
                           preferred_element_type=F32)


def _mlp2(x, layers, tile):
    (w1, b1), (w2, b2) = layers
    r, din = x.shape
    dh, dout = w1.shape[1], w2.shape[1]
    grid = r // tile

    def body(x_ref, w1_ref, b1_ref, w2_ref, b2_ref, o_ref):
        h = _sp(_dot(x_ref[...], w1_ref[...]) + b1_ref[...])
        o_ref[...] = _sp(_dot(h, w2_ref[...]) + b2_ref[...])

    return pl.pallas_call(
        body,
        grid=(grid,),
        in_specs=[
            pl.BlockSpec((tile, din), lambda i: (i, 0)),
            pl.BlockSpec((din, dh), lambda i: (0, 0)),
            pl.BlockSpec((1, dh), lambda i: (0, 0)),
            pl.BlockSpec((dh, dout), lambda i: (0, 0)),
            pl.BlockSpec((1, dout), lambda i: (0, 0)),
        ],
        out_specs=pl.BlockSpec((tile, dout), lambda i: (i, 0)),
        out_shape=jax.ShapeDtypeStruct((r, dout), F32),
    )(x, w1, b1.reshape(1, -1), w2, b2.reshape(1, -1))


def _edge_conv(ef_in, gsrc, gdst, u_cur, dense_layers, conv_layers, tile):
    (w1, b1), (w2, b2), (w3, b3) = conv_layers
    d = gsrc.shape[1]
    ws, wd, we, wu = w1[:d], w1[d:2 * d], w1[2 * d:3 * d], w1[3 * d:]
    e = ef_in.shape[0]
    grid = e // tile
    has_dense = dense_layers is not None
    weights = [ws, wd, we, wu, b1.reshape(1, -1), w2, b2.reshape(1, -1),
               w3, b3.reshape(1, -1)]
    if has_dense:
        (dw1, db1), (dw2, db2) = dense_layers
        weights += [dw1, db1.reshape(1, -1), dw2, db2.reshape(1, -1)]

    def body(ef_ref, gs_ref, gd_ref, u_ref, *refs):
        wrefs = refs[:len(weights)]
        e2_ref, res_ref = refs[len(weights):]
        ws_r, wd_r, we_r, wu_r, b1_r, w2_r, b2_r, w3_r, b3_r = wrefs[:9]
        x = ef_ref[...]
        xin = x
        if has_dense:
            dw1_r, db1_r, dw2_r, db2_r = wrefs[9:]
            x = _sp(_dot(x, dw1_r[...]) + db1_r[...])
            x = _sp(_dot(x, dw2_r[...]) + db2_r[...])
        h = (_dot(gs_ref[...], ws_r[...]) + _dot(gd_ref[...], wd_r[...])
             + _dot(x, we_r[...]) + _dot(u_ref[...], wu_r[...]) + b1_r[...])
        h = _sp(h)
        h = _sp(_dot(h, w2_r[...]) + b2_r[...])
        e2 = _sp(_dot(h, w3_r[...]) + b3_r[...])
        e2_ref[...] = e2
        res_ref[...] = e2 + xin

    row_spec = pl.BlockSpec((tile, d), lambda i: (i, 0))
    w_specs = [pl.BlockSpec(w.shape, lambda i: (0, 0)) for w in weights]
    dout = w3.shape[1]
    return pl.pallas_call(
        body,
        grid=(grid,),
        in_specs=[row_spec, row_spec, row_spec,
                  pl.BlockSpec((1, d), lambda i: (0, 0))] + w_specs,
        out_specs=[pl.BlockSpec((tile, dout), lambda i: (i, 0))] * 2,
        out_shape=[jax.ShapeDtypeStruct((e, dout), F32)] * 2,
    )(ef_in, gsrc, gdst, u_cur, *weights)


def _node_conv(nf_cur, nf_in, pa, pb, ca, cb, u_cur, conv_layers, tile):
    (w1, b1), (w2, b2), (w3, b3) = conv_layers
    d = nf_cur.shape[1]
    wn, we, wu = w1[:d], w1[d:2 * d], w1[2 * d:]
    n = nf_cur.shape[0]
    grid = n // tile
    weights = [wn, we, wu, b1.reshape(1, -1), w2, b2.reshape(1, -1),
               w3, b3.reshape(1, -1)]

    def body(nc_ref, ni_ref, pa_ref, pb_ref, ca_ref, cb_ref, u_ref, *refs):
        wn_r, we_r, wu_r, b1_r, w2_r, b2_r, w3_r, b3_r = refs[:8]
        n2_ref, res_ref = refs[8:]
        emean = (pa_ref[...] + pb_ref[...]) / jnp.maximum(
            ca_ref[...] + cb_ref[...], 1.0)
        h = (_dot(nc_ref[...], wn_r[...]) + _dot(emean, we_r[...])
             + _dot(u_ref[...], wu_r[...]) + b1_r[...])
        h = _sp(h)
        h = _sp(_dot(h, w2_r[...]) + b2_r[...])
        n2 = _sp(_dot(h, w3_r[...]) + b3_r[...])
        n2_ref[...] = n2
        res_ref[...] = n2 + ni_ref[...]

    row_spec = pl.BlockSpec((tile, d), lambda i: (i, 0))
    w_specs = [pl.BlockSpec(w.shape, lambda i: (0, 0)) for w in weights]
    dout = w3.shape[1]
    return pl.pallas_call(
        body,
        grid=(grid,),
        in_specs=[row_spec] * 6 + [pl.BlockSpec((1, d), lambda i: (0, 0))]
        + w_specs,
        out_specs=[pl.BlockSpec((tile, dout), lambda i: (i, 0))] * 2,
        out_shape=[jax.ShapeDtypeStruct((n, dout), F32)] * 2,
    )(nf_cur, nf_in, pa, pb, ca, cb, u_cur, *weights)


def _attr_conv(pa, pb, nf2, u_cur, u_in, conv_layers, n_edges):
    (w1, b1), (w2, b2), (w3, b3) = conv_layers
    d = u_cur.shape[1]
    we, wn, wu = w1[:d], w1[d:2 * d], w1[2 * d:]
    n = nf2.shape[0]
    weights = [we, wn, wu, b1.reshape(1, -1), w2, b2.reshape(1, -1),
               w3, b3.reshape(1, -1)]

    def body(pa_ref, pb_ref, nf2_ref, uc_ref, ui_ref, *refs):
        we_r, wn_r, wu_r, b1_r, w2_r, b2_r, w3_r, b3_r = refs[:8]
        o_ref = refs[8]
        mean_ef = jnp.sum(pa_ref[...] + pb_ref[...], axis=0,
                          keepdims=True) * (1.0 / n_edges)
        mean_nf = jnp.sum(nf2_ref[...], axis=0, keepdims=True) * (1.0 / n)
        h = (_dot(mean_ef, we_r[...]) + _dot(mean_nf, wn_r[...])
             + _dot(uc_ref[...], wu_r[...]) + b1_r[...])
        h = _sp(h)
        h = _sp(_dot(h, w2_r[...]) + b2_r[...])
        o_ref[...] = _sp(_dot(h, w3_r[...]) + b3_r[...]) + ui_ref[...]

    return pl.pallas_call(
        body,
        out_shape=jax.ShapeDtypeStruct((1, w3.shape[1]), F32),
    )(pa, pb, nf2, u_cur, u_in, *weights)


def _set2set(feat, p, tile):
    wi, wh, b = p['Wi'], p['Wh'], p['b']
    r, d = feat.shape
    ntiles = r // tile

    def body(f_ref, wi_ref, wh_ref, b_ref, q_ref,
             h_ref, c_ref, qs_ref, r_ref, m_ref, s_ref):
        it = pl.program_id(0)
        t = pl.program_id(1)

        @pl.when(t == 0)
        def _start_iter():
            @pl.when(it == 0)
            def _init():
                h_ref[...] = jnp.zeros((1, d), F32)
                c_ref[...] = jnp.zeros((1, d), F32)
                qs_ref[...] = jnp.zeros((1, 2 * d), F32)

            gates = (_dot(qs_ref[...], wi_ref[...])
                     + _dot(h_ref[...], wh_ref[...]) + b_ref[...])
            gi = _sig(gates[:, :d])
            gf = _sig(gates[:, d:2 * d])
            gg = jnp.tanh(gates[:, 2 * d:3 * d])
            go = _sig(gates[:, 3 * d:])
            c = gf * c_ref[...] + gi * gg
            c_ref[...] = c
            h_ref[...] = go * jnp.tanh(c)
            r_ref[...] = jnp.zeros((1, d), F32)
            m_ref[0, 0] = -1e30
            s_ref[0, 0] = 0.0

        f = f_ref[...]
        h = h_ref[...]
        logits = lax.dot_general(f, h, (((1,), (1,)), ((), ())),
                                 preferred_element_type=F32)
        m_old = m_ref[0, 0]
        m_new = jnp.maximum(m_old, jnp.max(logits))
        corr = jnp.exp(m_old - m_new)
        ex = jnp.exp(logits - m_new)
        s_ref[0, 0] = s_ref[0, 0] * corr + jnp.sum(ex)
        r_ref[...] = r_ref[...] * corr + lax.dot_general(
            ex, f, (((0,), (0,)), ((), ())), preferred_element_type=F32)
        m_ref[0, 0] = m_new

        @pl.when(t == ntiles - 1)
        def _end_iter():
            rvec = r_ref[...] / s_ref[0, 0]
            q = jnp.concatenate([h_ref[...], rvec], axis=1)
            qs_ref[...] = q

            @pl.when(it == 2)
            def _emit():
                q_ref[...] = q

    return pl.pallas_call(
        body,
        grid=(3, ntiles),
        in_specs=[
            pl.BlockSpec((tile, d), lambda it, t: (t, 0)),
            pl.BlockSpec(wi.shape, lambda it, t: (0, 0)),
            pl.BlockSpec(wh.shape, lambda it, t: (0, 0)),
            pl.BlockSpec((1, 4 * d), lambda it, t: (0, 0)),
        ],
        out_specs=pl.BlockSpec((1, 2 * d), lambda it, t: (0, 0)),
        out_shape=jax.ShapeDtypeStruct((1, 2 * d), F32),
        scratch_shapes=[
            pltpu.VMEM((1, d), F32), pltpu.VMEM((1, d), F32),
            pltpu.VMEM((1, 2 * d), F32), pltpu.VMEM((1, d), F32),
            pltpu.SMEM((1, 1), F32), pltpu.SMEM((1, 1), F32),
        ],
    )(feat, wi, wh, b.reshape(1, -1))


def _out_head(nq, eq, u, layers):
    (w1, b1), (w2, b2), (w3, b3) = layers
    dq = nq.shape[1]
    wn, we, wu = w1[:dq], w1[dq:2 * dq], w1[2 * dq:]
    weights = [wn, we, wu, b1.reshape(1, -1), w2, b2.reshape(1, -1),
               w3, b3.reshape(1, -1)]

    def body(nq_ref, eq_ref, u_ref, *refs):
        wn_r, we_r, wu_r, b1_r, w2_r, b2_r, w3_r, b3_r = refs[:8]
        o_ref = refs[8]
        h = (_dot(nq_ref[...], wn_r[...]) + _dot(eq_ref[...], we_r[...])
             + _dot(u_ref[...], wu_r[...]) + b1_r[...])
        h = _sp(h)
        h = _sp(_dot(h, w2_r[...]) + b2_r[...])
        o_ref[...] = _sig(_dot(h, w3_r[...]) + b3_r[...])

    return pl.pallas_call(
        body,
        out_shape=jax.ShapeDtypeStruct((1, 1), F32),
    )(nq, eq, u, *weights)


def _sc_mesh():
    return plsc.VectorSubcoreMesh(core_axis_name="c", subcore_axis_name="s")


_SC_PARAMS = pltpu.CompilerParams(use_tc_tiling_on_sc=False)


def _sc_gather2(table, src, dst):
    n, d = table.shape
    e = src.shape[0]
    n_groups = e // _G

    @functools.partial(
        pl.kernel, mesh=_sc_mesh(),
        out_type=[jax.ShapeDtypeStruct((e, d), F32),
                  jax.ShapeDtypeStruct((e, d), F32)],
        compiler_params=_SC_PARAMS,
        scratch_types=[
            pltpu.VMEM((_G,), jnp.int32), pltpu.VMEM((_G,), jnp.int32),
            pltpu.VMEM((_G, d), F32), pltpu.VMEM((_G, d), F32),
            pltpu.SemaphoreType.DMA, pltpu.SemaphoreType.DMA,
        ],
    )
    def k(table_hbm, src_hbm, dst_hbm, gs_hbm, gd_hbm,
          idx_s, idx_d, row_s, row_d, sem_s, sem_d):
        wid = lax.axis_index("s") * 2 + lax.axis_index("c")

        def body(i, carry):
            base = (wid + i * _NW) * _G
            pltpu.sync_copy(src_hbm.at[pl.ds(base, _G)], idx_s)
            pltpu.sync_copy(dst_hbm.at[pl.ds(base, _G)], idx_d)
            cp_s = pltpu.async_copy(table_hbm.at[idx_s], row_s, sem_s)
            cp_d = pltpu.async_copy(table_hbm.at[idx_d], row_d, sem_d)
            cp_s.wait()
            cp_d.wait()
            pltpu.sync_copy(row_s, gs_hbm.at[pl.ds(base, _G)])
            pltpu.sync_copy(row_d, gd_hbm.at[pl.ds(base, _G)])
            return carry

        n_mine = (n_groups - wid + _NW - 1) // _NW
        lax.fori_loop(0, n_mine, body, 0)

    return k(table, src, dst)


def _sc_scatter(vals, dst, n):
    e, d = vals.shape
    n_groups = e // _G
    zeros = jnp.zeros((n, d), F32)

    @functools.partial(
        pl.kernel, mesh=_sc_mesh(),
        out_type=jax.ShapeDtypeStruct((2, n, d), F32),
        compiler_params=_SC_PARAMS,
        scratch_types=[
            pltpu.VMEM((_G,), jnp.int32), pltpu.VMEM((_G, d), F32),
            pltpu.VMEM_SHARED((n, d), F32),
        ],
    )
    def k(vals_hbm, dst_hbm, zeros_hbm, out_hbm, idx_v, val_v, acc):
        cid = lax.axis_index("c")
        sid = lax.axis_index("s")
        wid = sid * 2 + cid

        @pl.when(sid == 0)
        def _():
            pltpu.sync_copy(zeros_hbm, acc)

        plsc.subcore_barrier()

        def body(i, carry):
            base = (wid + i * _NW) * _G
            pltpu.sync_copy(dst_hbm.at[pl.ds(base, _G)], idx_v)
            pltpu.sync_copy(vals_hbm.at[pl.ds(base, _G)], val_v)
            pltpu.sync_copy(val_v, acc.at[idx_v], add=True)
            return carry

        n_mine = (n_groups - wid + _NW - 1) // _NW
        lax.fori_loop(0, n_mine, body, 0)
        plsc.subcore_barrier()
        rows = n // 16
        pltpu.sync_copy(acc.at[pl.ds(sid * rows, rows)],
                        out_hbm.at[cid].at[pl.ds(sid * rows, rows)])

    return k(vals, dst, zeros)


def _sc_count(dst, n, d):
    e = dst.shape[0]
    n_groups = e // _G
    zeros = jnp.zeros((n, d), F32)
    ones = jnp.ones((_G, d), F32)

    @functools.partial(
        pl.kernel, mesh=_sc_mesh(),
        out_type=jax.ShapeDtypeStruct((2, n, d), F32),
        compiler_params=_SC_PARAMS,
        scratch_types=[
            pltpu.VMEM((_G,), jnp.int32), pltpu.VMEM((_G, d), F32),
            pltpu.VMEM_SHARED((n, d), F32),
        ],
    )
    def k(dst_hbm, zeros_hbm, ones_hbm, out_hbm, idx_v, one_v, acc):
        cid = lax.axis_index("c")
        sid = lax.axis_index("s")
        wid = sid * 2 + cid

        @pl.when(sid == 0)
        def _():
            pltpu.sync_copy(zeros_hbm, acc)

        pltpu.sync_copy(ones_hbm, one_v)
        plsc.subcore_barrier()

        def body(i, carry):
            base = (wid + i * _NW) * _G
            pltpu.sync_copy(dst_hbm.at[pl.ds(base, _G)], idx_v)
            pltpu.sync_copy(one_v, acc.at[idx_v], add=True)
            return carry

        n_mine = (n_groups - wid + _NW - 1) // _NW
        lax.fori_loop(0, n_mine, body, 0)
        plsc.subcore_barrier()
        rows = n // 16
        pltpu.sync_copy(acc.at[pl.ds(sid * rows, rows)],
                        out_hbm.at[cid].at[pl.ds(sid * rows, rows)])

    return k(dst, zeros, ones)


def kernel(edge_index, edge_feat, node_feat, graph_attr, params):
    src, dst = edge_index[0], edge_index[1]
    p = params
    n_nodes = node_feat.shape[0]
    n_edges = edge_feat.shape[0]

    ef = _mlp2(edge_feat, p['edge_enc'], tile=5000)
    nf = _mlp2(node_feat, p['node_enc'], tile=2000)
    u = _mlp2(graph_attr, p['attr_enc'], tile=1)

    d = ef.shape[1]
    cnt = _sc_count(dst, n_nodes, d)
    ca, cb = cnt[0], cnt[1]

    for blk in p['blocks']:
        ef_in, nf_in, u_in = ef, nf, u
        if blk['dense'] is not None:
            nf_cur = _mlp2(nf, blk['dense']['node'], tile=2000)
            u_cur = _mlp2(u, blk['dense']['attr'], tile=1)
            dense_edge = blk['dense']['edge']
        else:
            nf_cur, u_cur, dense_edge = nf, u, None
        gsrc, gdst = _sc_gather2(nf_cur, src, dst)
        ef2, ef = _edge_conv(ef_in, gsrc, gdst, u_cur, dense_edge,
                             blk['conv']['edge'], tile=5000)
        ps = _sc_scatter(ef2, dst, n_nodes)
        nf2, nf = _node_conv(nf_cur, nf_in, ps[0], ps[1], ca, cb, u_cur,
                             blk['conv']['node'], tile=2000)
        u = _attr_conv(ps[0], ps[1], nf2, u_cur, u_in,
                       blk['conv']['attr'], n_edges)

    nq = _set2set(nf, p['node_s2s'], tile=10000)
    eq = _set2set(ef, p['edge_s2s'], tile=10000)
    return _out_head(nq, eq, u, p['out'])


# speedup vs baseline: 2.2356x; 1.1104x over previous
import functools

import jax
import jax.numpy as jnp
from jax import lax
from jax.experimental import pallas as pl
from jax.experimental.pallas import tpu as pltpu
from jax.experimental.pallas import tpu_sc as plsc

F32 = jnp.float32
_G = 128
_NW = 32


def _sp(x):
    return jnp.maximum(x, 0.0) + jnp.log(1.0 + jnp.exp(-jnp.abs(x)))


def _sig(x):
    return 1.0 / (1.0 + jnp.exp(-x))


def _dot(a, b):
    return lax.dot_general(a, b, (((1,), (0,)), ((), ())),
                           preferred_element_type=F32)


def _mlp2(x, layers, tile):
    (w1, b1), (w2, b2) = layers
    r, din = x.shape
    dh, dout = w1.shape[1], w2.shape[1]
    grid = r // tile

    def body(x_ref, w1_ref, b1_ref, w2_ref, b2_ref, o_ref):
        h = _sp(_dot(x_ref[...], w1_ref[...]) + b1_ref[...])
        o_ref[...] = _sp(_dot(h, w2_ref[...]) + b2_ref[...])

    return pl.pallas_call(
        body,
        grid=(grid,),
        in_specs=[
            pl.BlockSpec((tile, din), lambda i: (i, 0)),
            pl.BlockSpec((din, dh), lambda i: (0, 0)),
            pl.BlockSpec((1, dh), lambda i: (0, 0)),
            pl.BlockSpec((dh, dout), lambda i: (0, 0)),
            pl.BlockSpec((1, dout), lambda i: (0, 0)),
        ],
        out_specs=pl.BlockSpec((tile, dout), lambda i: (i, 0)),
        out_shape=jax.ShapeDtypeStruct((r, dout), F32),
    )(x, w1, b1.reshape(1, -1), w2, b2.reshape(1, -1))


def _edge_conv(ef_in, gsrc, gdst, u_cur, dense_layers, conv_layers, tile):
    (w1, b1), (w2, b2), (w3, b3) = conv_layers
    d = gsrc.shape[1]
    ws, wd, we, wu = w1[:d], w1[d:2 * d], w1[2 * d:3 * d], w1[3 * d:]
    e = ef_in.shape[0]
    grid = e // tile
    has_dense = dense_layers is not None
    weights = [ws, wd, we, wu, b1.reshape(1, -1), w2, b2.reshape(1, -1),
               w3, b3.reshape(1, -1)]
    if has_dense:
        (dw1, db1), (dw2, db2) = dense_layers
        weights += [dw1, db1.reshape(1, -1), dw2, db2.reshape(1, -1)]

    def body(ef_ref, gs_ref, gd_ref, u_ref, *refs):
        wrefs = refs[:len(weights)]
        e2_ref, res_ref = refs[len(weights):]
        ws_r, wd_r, we_r, wu_r, b1_r, w2_r, b2_r, w3_r, b3_r = wrefs[:9]
        x = ef_ref[...]
        xin = x
        if has_dense:
            dw1_r, db1_r, dw2_r, db2_r = wrefs[9:]
            x = _sp(_dot(x, dw1_r[...]) + db1_r[...])
            x = _sp(_dot(x, dw2_r[...]) + db2_r[...])
        h = (_dot(gs_ref[...], ws_r[...]) + _dot(gd_ref[...], wd_r[...])
             + _dot(x, we_r[...]) + _dot(u_ref[...], wu_r[...]) + b1_r[...])
        h = _sp(h)
        h = _sp(_dot(h, w2_r[...]) + b2_r[...])
        e2 = _sp(_dot(h, w3_r[...]) + b3_r[...])
        e2_ref[...] = e2
        res_ref[...] = e2 + xin

    row_spec = pl.BlockSpec((tile, d), lambda i: (i, 0))
    w_specs = [pl.BlockSpec(w.shape, lambda i: (0, 0)) for w in weights]
    dout = w3.shape[1]
    return pl.pallas_call(
        body,
        grid=(grid,),
        in_specs=[row_spec, row_spec, row_spec,
                  pl.BlockSpec((1, d), lambda i: (0, 0))] + w_specs,
        out_specs=[pl.BlockSpec((tile, dout), lambda i: (i, 0))] * 2,
        out_shape=[jax.ShapeDtypeStruct((e, dout), F32)] * 2,
    )(ef_in, gsrc, gdst, u_cur, *weights)


def _node_conv(nf_cur, nf_in, pa, pb, ca, cb, u_cur, conv_layers, tile):
    (w1, b1), (w2, b2), (w3, b3) = conv_layers
    d = nf_cur.shape[1]
    wn, we, wu = w1[:d], w1[d:2 * d], w1[2 * d:]
    n = nf_cur.shape[0]
    grid = n // tile
    weights = [wn, we, wu, b1.reshape(1, -1), w2, b2.reshape(1, -1),
               w3, b3.reshape(1, -1)]

    def body(nc_ref, ni_ref, pa_ref, pb_ref, ca_ref, cb_ref, u_ref, *refs):
        wn_r, we_r, wu_r, b1_r, w2_r, b2_r, w3_r, b3_r = refs[:8]
        n2_ref, res_ref = refs[8:]
        emean = (pa_ref[...] + pb_ref[...]) / jnp.maximum(
            ca_ref[...] + cb_ref[...], 1.0)
        h = (_dot(nc_ref[...], wn_r[...]) + _dot(emean, we_r[...])
             + _dot(u_ref[...], wu_r[...]) + b1_r[...])
        h = _sp(h)
        h = _sp(_dot(h, w2_r[...]) + b2_r[...])
        n2 = _sp(_dot(h, w3_r[...]) + b3_r[...])
        n2_ref[...] = n2
        res_ref[...] = n2 + ni_ref[...]

    row_spec = pl.BlockSpec((tile, d), lambda i: (i, 0))
    w_specs = [pl.BlockSpec(w.shape, lambda i: (0, 0)) for w in weights]
    dout = w3.shape[1]
    return pl.pallas_call(
        body,
        grid=(grid,),
        in_specs=[row_spec] * 6 + [pl.BlockSpec((1, d), lambda i: (0, 0))]
        + w_specs,
        out_specs=[pl.BlockSpec((tile, dout), lambda i: (i, 0))] * 2,
        out_shape=[jax.ShapeDtypeStruct((n, dout), F32)] * 2,
    )(nf_cur, nf_in, pa, pb, ca, cb, u_cur, *weights)


def _attr_conv(pa, pb, nf2, u_cur, u_in, conv_layers, n_edges):
    (w1, b1), (w2, b2), (w3, b3) = conv_layers
    d = u_cur.shape[1]
    we, wn, wu = w1[:d], w1[d:2 * d], w1[2 * d:]
    n = nf2.shape[0]
    weights = [we, wn, wu, b1.reshape(1, -1), w2, b2.reshape(1, -1),
               w3, b3.reshape(1, -1)]

    def body(pa_ref, pb_ref, nf2_ref, uc_ref, ui_ref, *refs):
        we_r, wn_r, wu_r, b1_r, w2_r, b2_r, w3_r, b3_r = refs[:8]
        o_ref = refs[8]
        mean_ef = jnp.sum(pa_ref[...] + pb_ref[...], axis=0,
                          keepdims=True) * (1.0 / n_edges)
        mean_nf = jnp.sum(nf2_ref[...], axis=0, keepdims=True) * (1.0 / n)
        h = (_dot(mean_ef, we_r[...]) + _dot(mean_nf, wn_r[...])
             + _dot(uc_ref[...], wu_r[...]) + b1_r[...])
        h = _sp(h)
        h = _sp(_dot(h, w2_r[...]) + b2_r[...])
        o_ref[...] = _sp(_dot(h, w3_r[...]) + b3_r[...]) + ui_ref[...]

    return pl.pallas_call(
        body,
        out_shape=jax.ShapeDtypeStruct((1, w3.shape[1]), F32),
    )(pa, pb, nf2, u_cur, u_in, *weights)


def _set2set(feat, p, tile):
    wi, wh, b = p['Wi'], p['Wh'], p['b']
    r, d = feat.shape
    ntiles = r // tile

    def body(f_ref, wi_ref, wh_ref, b_ref, q_ref,
             h_ref, c_ref, qs_ref, r_ref, m_ref, s_ref):
        it = pl.program_id(0)
        t = pl.program_id(1)

        @pl.when(t == 0)
        def _start_iter():
            @pl.when(it == 0)
            def _init():
                h_ref[...] = jnp.zeros((1, d), F32)
                c_ref[...] = jnp.zeros((1, d), F32)
                qs_ref[...] = jnp.zeros((1, 2 * d), F32)

            gates = (_dot(qs_ref[...], wi_ref[...])
                     + _dot(h_ref[...], wh_ref[...]) + b_ref[...])
            gi = _sig(gates[:, :d])
            gf = _sig(gates[:, d:2 * d])
            gg = jnp.tanh(gates[:, 2 * d:3 * d])
            go = _sig(gates[:, 3 * d:])
            c = gf * c_ref[...] + gi * gg
            c_ref[...] = c
            h_ref[...] = go * jnp.tanh(c)
            r_ref[...] = jnp.zeros((1, d), F32)
            m_ref[0, 0] = -1e30
            s_ref[0, 0] = 0.0

        f = f_ref[...]
        h = h_ref[...]
        logits = lax.dot_general(f, h, (((1,), (1,)), ((), ())),
                                 preferred_element_type=F32)
        m_old = m_ref[0, 0]
        m_new = jnp.maximum(m_old, jnp.max(logits))
        corr = jnp.exp(m_old - m_new)
        ex = jnp.exp(logits - m_new)
        s_ref[0, 0] = s_ref[0, 0] * corr + jnp.sum(ex)
        r_ref[...] = r_ref[...] * corr + lax.dot_general(
            ex, f, (((0,), (0,)), ((), ())), preferred_element_type=F32)
        m_ref[0, 0] = m_new

        @pl.when(t == ntiles - 1)
        def _end_iter():
            rvec = r_ref[...] / s_ref[0, 0]
            q = jnp.concatenate([h_ref[...], rvec], axis=1)
            qs_ref[...] = q

            @pl.when(it == 2)
            def _emit():
                q_ref[...] = q

    return pl.pallas_call(
        body,
        grid=(3, ntiles),
        in_specs=[
            pl.BlockSpec((tile, d), lambda it, t: (t, 0)),
            pl.BlockSpec(wi.shape, lambda it, t: (0, 0)),
            pl.BlockSpec(wh.shape, lambda it, t: (0, 0)),
            pl.BlockSpec((1, 4 * d), lambda it, t: (0, 0)),
        ],
        out_specs=pl.BlockSpec((1, 2 * d), lambda it, t: (0, 0)),
        out_shape=jax.ShapeDtypeStruct((1, 2 * d), F32),
        scratch_shapes=[
            pltpu.VMEM((1, d), F32), pltpu.VMEM((1, d), F32),
            pltpu.VMEM((1, 2 * d), F32), pltpu.VMEM((1, d), F32),
            pltpu.SMEM((1, 1), F32), pltpu.SMEM((1, 1), F32),
        ],
    )(feat, wi, wh, b.reshape(1, -1))


def _out_head(nq, eq, u, layers):
    (w1, b1), (w2, b2), (w3, b3) = layers
    dq = nq.shape[1]
    wn, we, wu = w1[:dq], w1[dq:2 * dq], w1[2 * dq:]
    weights = [wn, we, wu, b1.reshape(1, -1), w2, b2.reshape(1, -1),
               w3, b3.reshape(1, -1)]

    def body(nq_ref, eq_ref, u_ref, *refs):
        wn_r, we_r, wu_r, b1_r, w2_r, b2_r, w3_r, b3_r = refs[:8]
        o_ref = refs[8]
        h = (_dot(nq_ref[...], wn_r[...]) + _dot(eq_ref[...], we_r[...])
             + _dot(u_ref[...], wu_r[...]) + b1_r[...])
        h = _sp(h)
        h = _sp(_dot(h, w2_r[...]) + b2_r[...])
        o_ref[...] = _sig(_dot(h, w3_r[...]) + b3_r[...])

    return pl.pallas_call(
        body,
        out_shape=jax.ShapeDtypeStruct((1, 1), F32),
    )(nq, eq, u, *weights)


def _sc_mesh():
    return plsc.VectorSubcoreMesh(core_axis_name="c", subcore_axis_name="s")


_SC_PARAMS = pltpu.CompilerParams(use_tc_tiling_on_sc=False)


def _worker_span(n_groups):
    base_cnt = n_groups // _NW
    rem = n_groups - base_cnt * _NW

    def span(wid):
        n_mine = base_cnt + (wid < rem).astype(jnp.int32)
        row0 = wid * base_cnt + jnp.minimum(wid, rem)
        return row0, n_mine

    return base_cnt, rem, span


def _load_idx(src2, idx_v, row0, n_mine, base_cnt):
    pltpu.sync_copy(src2.at[pl.ds(row0, base_cnt)],
                    idx_v.at[pl.ds(0, base_cnt)])

    @pl.when(n_mine > base_cnt)
    def _():
        pltpu.sync_copy(src2.at[pl.ds(row0 + base_cnt, 1)],
                        idx_v.at[pl.ds(base_cnt, 1)])


def _sc_gather2(table, src2, dst2):
    n, d = table.shape
    n_groups = src2.shape[0]
    e = n_groups * _G
    base_cnt, rem, span = _worker_span(n_groups)
    max_cnt = base_cnt + (1 if rem else 0)
    n_outer = (max_cnt + 1) // 2 + 1

    @functools.partial(
        pl.kernel, mesh=_sc_mesh(),
        out_type=[jax.ShapeDtypeStruct((e, d), F32),
                  jax.ShapeDtypeStruct((e, d), F32)],
        compiler_params=_SC_PARAMS,
        scratch_types=[
            pltpu.VMEM((max_cnt, _G), jnp.int32),
            pltpu.VMEM((max_cnt, _G), jnp.int32),
            pltpu.VMEM((_G, d), F32), pltpu.VMEM((_G, d), F32),
            pltpu.VMEM((_G, d), F32), pltpu.VMEM((_G, d), F32),
        ] + [pltpu.SemaphoreType.DMA] * 8,
    )
    def k(table_hbm, src_hbm, dst_hbm, gs_hbm, gd_hbm,
          idx_s, idx_d, rs0, rs1, rd0, rd1, *sems):
        rows_s, rows_d = (rs0, rs1), (rd0, rd1)
        sem_gs, sem_gd = sems[0:2], sems[2:4]
        sem_os, sem_od = sems[4:6], sems[6:8]
        wid = lax.axis_index("s") * 2 + lax.axis_index("c")
        row0, n_mine = span(wid)
        _load_idx(src_hbm, idx_s, row0, n_mine, base_cnt)
        _load_idx(dst_hbm, idx_d, row0, n_mine, base_cnt)

        def body(i, carry):
            for b in (0, 1):
                g = i * 2 + b
                ok = g < n_mine

                @pl.when(jnp.logical_and(ok, g >= 2))
                def _drain_writes():
                    base = (row0 + g) * _G
                    pltpu.make_async_copy(
                        rows_s[b], gs_hbm.at[pl.ds(base, _G)],
                        sem_os[b]).wait()
                    pltpu.make_async_copy(
                        rows_d[b], gd_hbm.at[pl.ds(base, _G)],
                        sem_od[b]).wait()

                @pl.when(ok)
                def _issue_gathers():
                    pltpu.async_copy(table_hbm.at[idx_s.at[g]], rows_s[b],
                                     sem_gs[b])
                    pltpu.async_copy(table_hbm.at[idx_d.at[g]], rows_d[b],
                                     sem_gd[b])

                gp = g - 1
                bp = 1 - b

                @pl.when(jnp.logical_and(gp >= 0, gp < n_mine))
                def _complete_prev():
                    pltpu.make_async_copy(table_hbm.at[idx_s.at[gp]],
                                          rows_s[bp], sem_gs[bp]).wait()
                    pltpu.make_async_copy(table_hbm.at[idx_d.at[gp]],
                                          rows_d[bp], sem_gd[bp]).wait()
                    base = (row0 + gp) * _G
                    pltpu.async_copy(rows_s[bp], gs_hbm.at[pl.ds(base, _G)],
                                     sem_os[bp])
                    pltpu.async_copy(rows_d[bp], gd_hbm.at[pl.ds(base, _G)],
                                     sem_od[bp])
            return carry

        lax.fori_loop(0, n_outer, body, 0)
        for b in (0, 1):
            base = row0 * _G
            pltpu.make_async_copy(rows_s[b], gs_hbm.at[pl.ds(base, _G)],
                                  sem_os[b]).wait()
            pltpu.make_async_copy(rows_d[b], gd_hbm.at[pl.ds(base, _G)],
                                  sem_od[b]).wait()

    return k(table, src2, dst2)


def _sc_scatter(vals, dst2, n):
    e, d = vals.shape
    n_groups = dst2.shape[0]
    zeros = jnp.zeros((n, d), F32)
    base_cnt, rem, span = _worker_span(n_groups)
    max_cnt = base_cnt + (1 if rem else 0)
    n_outer = (max_cnt + 1) // 2 + 1

    @functools.partial(
        pl.kernel, mesh=_sc_mesh(),
        out_type=jax.ShapeDtypeStruct((2, n, d), F32),
        compiler_params=_SC_PARAMS,
        scratch_types=[
            pltpu.VMEM((max_cnt, _G), jnp.int32),
            pltpu.VMEM((_G, d), F32), pltpu.VMEM((_G, d), F32),
            pltpu.VMEM_SHARED((n, d), F32),
        ] + [pltpu.SemaphoreType.DMA] * 4,
    )
    def k(vals_hbm, dst_hbm, zeros_hbm, out_hbm, idx_d, v0, v1, acc, *sems):
        val_v = (v0, v1)
        sem_v, sem_sc = sems[0:2], sems[2:4]
        cid = lax.axis_index("c")
        sid = lax.axis_index("s")
        wid = sid * 2 + cid
        row0, n_mine = span(wid)

        @pl.when(sid == 0)
        def _():
            pltpu.sync_copy(zeros_hbm, acc)

        _load_idx(dst_hbm, idx_d, row0, n_mine, base_cnt)
        plsc.subcore_barrier()

        def body(i, carry):
            for b in (0, 1):
                g = i * 2 + b
                ok = g < n_mine

                @pl.when(jnp.logical_and(ok, g >= 2))
                def _drain_scatter():
                    pltpu.make_async_copy(val_v[b], acc.at[idx_d.at[g]],
                                          sem_sc[b]).wait()

                @pl.when(ok)
                def _issue_load():
                    base = (row0 + g) * _G
                    pltpu.async_copy(vals_hbm.at[pl.ds(base, _G)], val_v[b],
                                     sem_v[b])

                gp = g - 1
                bp = 1 - b

                @pl.when(jnp.logical_and(gp >= 0, gp < n_mine))
                def _scatter_prev():
                    base = (row0 + gp) * _G
                    pltpu.make_async_copy(vals_hbm.at[pl.ds(base, _G)],
                                          val_v[bp], sem_v[bp]).wait()
                    pltpu.async_copy(val_v[bp], acc.at[idx_d.at[gp]],
                                     sem_sc[bp], add=True)
            return carry

        lax.fori_loop(0, n_outer, body, 0)
        for b in (0, 1):
            pltpu.make_async_copy(val_v[b], acc.at[idx_d.at[0]],
                                  sem_sc[b]).wait()
        plsc.subcore_barrier()
        rows = n // 16
        pltpu.sync_copy(acc.at[pl.ds(sid * rows, rows)],
                        out_hbm.at[cid].at[pl.ds(sid * rows, rows)])

    return k(vals, dst2, zeros)


def _sc_count(dst2, n, d):
    n_groups = dst2.shape[0]
    zeros = jnp.zeros((n, d), F32)
    ones = jnp.ones((_G, d), F32)
    base_cnt, rem, span = _worker_span(n_groups)
    max_cnt = base_cnt + (1 if rem else 0)
    n_outer = (max_cnt + 1) // 2 + 1

    @functools.partial(
        pl.kernel, mesh=_sc_mesh(),
        out_type=jax.ShapeDtypeStruct((2, n, d), F32),
        compiler_params=_SC_PARAMS,
        scratch_types=[
            pltpu.VMEM((max_cnt, _G), jnp.int32),
            pltpu.VMEM((_G, d), F32),
            pltpu.VMEM_SHARED((n, d), F32),
        ] + [pltpu.SemaphoreType.DMA] * 2,
    )
    def k(dst_hbm, zeros_hbm, ones_hbm, out_hbm, idx_d, one_v, acc, *sems):
        cid = lax.axis_index("c")
        sid = lax.axis_index("s")
        wid = sid * 2 + cid
        row0, n_mine = span(wid)

        @pl.when(sid == 0)
        def _():
            pltpu.sync_copy(zeros_hbm, acc)

        cp = pltpu.async_copy(ones_hbm, one_v, sems[0])
        _load_idx(dst_hbm, idx_d, row0, n_mine, base_cnt)
        cp.wait()
        plsc.subcore_barrier()

        def body(i, carry):
            for b in (0, 1):
                g = i * 2 + b
                ok = g < n_mine

                @pl.when(jnp.logical_and(ok, g >= 2))
                def _drain():
                    pltpu.make_async_copy(one_v, acc.at[idx_d.at[g]],
                                          sems[b]).wait()

                @pl.when(ok)
                def _issue():
                    pltpu.async_copy(one_v, acc.at[idx_d.at[g]],
                                     sems[b], add=True)
            return carry

        lax.fori_loop(0, n_outer, body, 0)
        for b in (0, 1):
            pltpu.make_async_copy(one_v, acc.at[idx_d.at[0]],
                                  sems[b]).wait()
        plsc.subcore_barrier()
        rows = n // 16
        pltpu.sync_copy(acc.at[pl.ds(sid * rows, rows)],
                        out_hbm.at[cid].at[pl.ds(sid * rows, rows)])

    return k(dst2, zeros, ones)


def kernel(edge_index, edge_feat, node_feat, graph_attr, params):
    p = params
    n_nodes = node_feat.shape[0]
    n_edges = edge_feat.shape[0]
    src2 = edge_index[0].reshape(n_edges // _G, _G)
    dst2 = edge_index[1].reshape(n_edges // _G, _G)

    ef = _mlp2(edge_feat, p['edge_enc'], tile=5000)
    nf = _mlp2(node_feat, p['node_enc'], tile=2000)
    u = _mlp2(graph_attr, p['attr_enc'], tile=1)

    d = ef.shape[1]
    cnt = _sc_count(dst2, n_nodes, d)
    ca, cb = cnt[0], cnt[1]

    for blk in p['blocks']:
        ef_in, nf_in, u_in = ef, nf, u
        if blk['dense'] is not None:
            nf_cur = _mlp2(nf, blk['dense']['node'], tile=2000)
            u_cur = _mlp2(u, blk['dense']['attr'], tile=1)
            dense_edge = blk['dense']['edge']
        else:
            nf_cur, u_cur, dense_edge = nf, u, None
        gsrc, gdst = _sc_gather2(nf_cur, src2, dst2)
        ef2, ef = _edge_conv(ef_in, gsrc, gdst, u_cur, dense_edge,
                             blk['conv']['edge'], tile=5000)
        ps = _sc_scatter(ef2, dst2, n_nodes)
        nf2, nf = _node_conv(nf_cur, nf_in, ps[0], ps[1], ca, cb, u_cur,
                             blk['conv']['node'], tile=2000)
        u = _attr_conv(ps[0], ps[1], nf2, u_cur, u_in,
                       blk['conv']['attr'], n_edges)

    nq = _set2set(nf, p['node_s2s'], tile=10000)
    eq = _set2set(ef, p['edge_s2s'], tile=10000)
    return _out_head(nq, eq, u, p['out'])


# speedup vs baseline: 2.4699x; 1.1048x over previous
import functools

import jax
import jax.numpy as jnp
from jax import lax
from jax.experimental import pallas as pl
from jax.experimental.pallas import tpu as pltpu
from jax.experimental.pallas import tpu_sc as plsc

F32 = jnp.float32
_G = 128
_NW = 32


def _sp(x):
    return jnp.maximum(x, 0.0) + jnp.log(1.0 + jnp.exp(-jnp.abs(x)))


def _sig(x):
    return 1.0 / (1.0 + jnp.exp(-x))


def _dot(a, b):
    return lax.dot_general(a, b, (((1,), (0,)), ((), ())),
                           preferred_element_type=F32)


def _mlp2(x, layers, tile):
    (w1, b1), (w2, b2) = layers
    r, din = x.shape
    dh, dout = w1.shape[1], w2.shape[1]
    grid = r // tile

    def body(x_ref, w1_ref, b1_ref, w2_ref, b2_ref, o_ref):
        h = _sp(_dot(x_ref[...], w1_ref[...]) + b1_ref[...])
        o_ref[...] = _sp(_dot(h, w2_ref[...]) + b2_ref[...])

    return pl.pallas_call(
        body,
        grid=(grid,),
        in_specs=[
            pl.BlockSpec((tile, din), lambda i: (i, 0)),
            pl.BlockSpec((din, dh), lambda i: (0, 0)),
            pl.BlockSpec((1, dh), lambda i: (0, 0)),
            pl.BlockSpec((dh, dout), lambda i: (0, 0)),
            pl.BlockSpec((1, dout), lambda i: (0, 0)),
        ],
        out_specs=pl.BlockSpec((tile, dout), lambda i: (i, 0)),
        out_shape=jax.ShapeDtypeStruct((r, dout), F32),
    )(x, w1, b1.reshape(1, -1), w2, b2.reshape(1, -1))


def _ab_tables(nf_cur, w1):
    d = nf_cur.shape[1]
    ws, wd = w1[:d], w1[d:2 * d]
    n = nf_cur.shape[0]
    tile = 2000
    dh = w1.shape[1]

    def body(nf_ref, ws_ref, wd_ref, a_ref, b_ref):
        x = nf_ref[...]
        a_ref[...] = _dot(x, ws_ref[...])
        b_ref[...] = _dot(x, wd_ref[...])

    return pl.pallas_call(
        body,
        grid=(n // tile,),
        in_specs=[pl.BlockSpec((tile, d), lambda i: (i, 0)),
                  pl.BlockSpec((d, dh), lambda i: (0, 0)),
                  pl.BlockSpec((d, dh), lambda i: (0, 0))],
        out_specs=[pl.BlockSpec((tile, dh), lambda i: (i, 0))] * 2,
        out_shape=[jax.ShapeDtypeStruct((n, dh), F32)] * 2,
    )(nf_cur, ws, wd)


def _edge_conv(ef_in, h1pre, u_cur, dense_layers, conv_layers, tile):
    (w1, b1), (w2, b2), (w3, b3) = conv_layers
    d = ef_in.shape[1]
    we, wu = w1[2 * d:3 * d], w1[3 * d:]
    dh = w1.shape[1]
    e = ef_in.shape[0]
    grid = e // tile
    has_dense = dense_layers is not None
    weights = [we, wu, b1.reshape(1, -1), w2, b2.reshape(1, -1),
               w3, b3.reshape(1, -1)]
    if has_dense:
        (dw1, db1), (dw2, db2) = dense_layers
        weights += [dw1, db1.reshape(1, -1), dw2, db2.reshape(1, -1)]

    def body(ef_ref, h1_ref, u_ref, *refs):
        wrefs = refs[:len(weights)]
        e2_ref, res_ref = refs[len(weights):]
        we_r, wu_r, b1_r, w2_r, b2_r, w3_r, b3_r = wrefs[:7]
        x = ef_ref[...]
        xin = x
        if has_dense:
            dw1_r, db1_r, dw2_r, db2_r = wrefs[7:]
            x = _sp(_dot(x, dw1_r[...]) + db1_r[...])
            x = _sp(_dot(x, dw2_r[...]) + db2_r[...])
        h = (h1_ref[...] + _dot(x, we_r[...])
             + _dot(u_ref[...], wu_r[...]) + b1_r[...])
        h = _sp(h)
        h = _sp(_dot(h, w2_r[...]) + b2_r[...])
        e2 = _sp(_dot(h, w3_r[...]) + b3_r[...])
        e2_ref[...] = e2
        res_ref[...] = e2 + xin

    w_specs = [pl.BlockSpec(w.shape, lambda i: (0, 0)) for w in weights]
    dout = w3.shape[1]
    return pl.pallas_call(
        body,
        grid=(grid,),
        in_specs=[pl.BlockSpec((tile, d), lambda i: (i, 0)),
                  pl.BlockSpec((tile, dh), lambda i: (i, 0)),
                  pl.BlockSpec((1, d), lambda i: (0, 0))] + w_specs,
        out_specs=[pl.BlockSpec((tile, dout), lambda i: (i, 0))] * 2,
        out_shape=[jax.ShapeDtypeStruct((e, dout), F32)] * 2,
    )(ef_in, h1pre, u_cur, *weights)


def _node_conv(nf_cur, nf_in, pa, pb, ca, cb, u_cur, conv_layers, tile):
    (w1, b1), (w2, b2), (w3, b3) = conv_layers
    d = nf_cur.shape[1]
    wn, we, wu = w1[:d], w1[d:2 * d], w1[2 * d:]
    n = nf_cur.shape[0]
    grid = n // tile
    weights = [wn, we, wu, b1.reshape(1, -1), w2, b2.reshape(1, -1),
               w3, b3.reshape(1, -1)]

    def body(nc_ref, ni_ref, pa_ref, pb_ref, ca_ref, cb_ref, u_ref, *refs):
        wn_r, we_r, wu_r, b1_r, w2_r, b2_r, w3_r, b3_r = refs[:8]
        n2_ref, res_ref = refs[8:]
        emean = (pa_ref[...] + pb_ref[...]) / jnp.maximum(
            ca_ref[...] + cb_ref[...], 1.0)
        h = (_dot(nc_ref[...], wn_r[...]) + _dot(emean, we_r[...])
             + _dot(u_ref[...], wu_r[...]) + b1_r[...])
        h = _sp(h)
        h = _sp(_dot(h, w2_r[...]) + b2_r[...])
        n2 = _sp(_dot(h, w3_r[...]) + b3_r[...])
        n2_ref[...] = n2
        res_ref[...] = n2 + ni_ref[...]

    row_spec = pl.BlockSpec((tile, d), lambda i: (i, 0))
    w_specs = [pl.BlockSpec(w.shape, lambda i: (0, 0)) for w in weights]
    dout = w3.shape[1]
    return pl.pallas_call(
        body,
        grid=(grid,),
        in_specs=[row_spec] * 6 + [pl.BlockSpec((1, d), lambda i: (0, 0))]
        + w_specs,
        out_specs=[pl.BlockSpec((tile, dout), lambda i: (i, 0))] * 2,
        out_shape=[jax.ShapeDtypeStruct((n, dout), F32)] * 2,
    )(nf_cur, nf_in, pa, pb, ca, cb, u_cur, *weights)


def _attr_conv(pa, pb, nf2, u_cur, u_in, conv_layers, n_edges):
    (w1, b1), (w2, b2), (w3, b3) = conv_layers
    d = u_cur.shape[1]
    we, wn, wu = w1[:d], w1[d:2 * d], w1[2 * d:]
    n = nf2.shape[0]
    weights = [we, wn, wu, b1.reshape(1, -1), w2, b2.reshape(1, -1),
               w3, b3.reshape(1, -1)]

    def body(pa_ref, pb_ref, nf2_ref, uc_ref, ui_ref, *refs):
        we_r, wn_r, wu_r, b1_r, w2_r, b2_r, w3_r, b3_r = refs[:8]
        o_ref = refs[8]
        mean_ef = jnp.sum(pa_ref[...] + pb_ref[...], axis=0,
                          keepdims=True) * (1.0 / n_edges)
        mean_nf = jnp.sum(nf2_ref[...], axis=0, keepdims=True) * (1.0 / n)
        h = (_dot(mean_ef, we_r[...]) + _dot(mean_nf, wn_r[...])
             + _dot(uc_ref[...], wu_r[...]) + b1_r[...])
        h = _sp(h)
        h = _sp(_dot(h, w2_r[...]) + b2_r[...])
        o_ref[...] = _sp(_dot(h, w3_r[...]) + b3_r[...]) + ui_ref[...]

    return pl.pallas_call(
        body,
        out_shape=jax.ShapeDtypeStruct((1, w3.shape[1]), F32),
    )(pa, pb, nf2, u_cur, u_in, *weights)


def _set2set(feat, p, tile):
    wi, wh, b = p['Wi'], p['Wh'], p['b']
    r, d = feat.shape
    ntiles = r // tile

    def body(f_ref, wi_ref, wh_ref, b_ref, q_ref,
             h_ref, c_ref, qs_ref, r_ref, m_ref, s_ref):
        it = pl.program_id(0)
        t = pl.program_id(1)

        @pl.when(t == 0)
        def _start_iter():
            @pl.when(it == 0)
            def _init():
                h_ref[...] = jnp.zeros((1, d), F32)
                c_ref[...] = jnp.zeros((1, d), F32)
                qs_ref[...] = jnp.zeros((1, 2 * d), F32)

            gates = (_dot(qs_ref[...], wi_ref[...])
                     + _dot(h_ref[...], wh_ref[...]) + b_ref[...])
            gi = _sig(gates[:, :d])
            gf = _sig(gates[:, d:2 * d])
            gg = jnp.tanh(gates[:, 2 * d:3 * d])
            go = _sig(gates[:, 3 * d:])
            c = gf * c_ref[...] + gi * gg
            c_ref[...] = c
            h_ref[...] = go * jnp.tanh(c)
            r_ref[...] = jnp.zeros((1, d), F32)
            m_ref[0, 0] = -1e30
            s_ref[0, 0] = 0.0

        f = f_ref[...]
        h = h_ref[...]
        logits = lax.dot_general(f, h, (((1,), (1,)), ((), ())),
                                 preferred_element_type=F32)
        m_old = m_ref[0, 0]
        m_new = jnp.maximum(m_old, jnp.max(logits))
        corr = jnp.exp(m_old - m_new)
        ex = jnp.exp(logits - m_new)
        s_ref[0, 0] = s_ref[0, 0] * corr + jnp.sum(ex)
        r_ref[...] = r_ref[...] * corr + lax.dot_general(
            ex, f, (((0,), (0,)), ((), ())), preferred_element_type=F32)
        m_ref[0, 0] = m_new

        @pl.when(t == ntiles - 1)
        def _end_iter():
            rvec = r_ref[...] / s_ref[0, 0]
            q = jnp.concatenate([h_ref[...], rvec], axis=1)
            qs_ref[...] = q

            @pl.when(it == 2)
            def _emit():
                q_ref[...] = q

    return pl.pallas_call(
        body,
        grid=(3, ntiles),
        in_specs=[
            pl.BlockSpec((tile, d), lambda it, t: (t, 0)),
            pl.BlockSpec(wi.shape, lambda it, t: (0, 0)),
            pl.BlockSpec(wh.shape, lambda it, t: (0, 0)),
            pl.BlockSpec((1, 4 * d), lambda it, t: (0, 0)),
        ],
        out_specs=pl.BlockSpec((1, 2 * d), lambda it, t: (0, 0)),
        out_shape=jax.ShapeDtypeStruct((1, 2 * d), F32),
        scratch_shapes=[
            pltpu.VMEM((1, d), F32), pltpu.VMEM((1, d), F32),
            pltpu.VMEM((1, 2 * d), F32), pltpu.VMEM((1, d), F32),
            pltpu.SMEM((1, 1), F32), pltpu.SMEM((1, 1), F32),
        ],
    )(feat, wi, wh, b.reshape(1, -1))


def _out_head(nq, eq, u, layers):
    (w1, b1), (w2, b2), (w3, b3) = layers
    dq = nq.shape[1]
    wn, we, wu = w1[:dq], w1[dq:2 * dq], w1[2 * dq:]
    weights = [wn, we, wu, b1.reshape(1, -1), w2, b2.reshape(1, -1),
               w3, b3.reshape(1, -1)]

    def body(nq_ref, eq_ref, u_ref, *refs):
        wn_r, we_r, wu_r, b1_r, w2_r, b2_r, w3_r, b3_r = refs[:8]
        o_ref = refs[8]
        h = (_dot(nq_ref[...], wn_r[...]) + _dot(eq_ref[...], we_r[...])
             + _dot(u_ref[...], wu_r[...]) + b1_r[...])
        h = _sp(h)
        h = _sp(_dot(h, w2_r[...]) + b2_r[...])
        o_ref[...] = _sig(_dot(h, w3_r[...]) + b3_r[...])

    return pl.pallas_call(
        body,
        out_shape=jax.ShapeDtypeStruct((1, 1), F32),
    )(nq, eq, u, *weights)


def _sc_mesh():
    return plsc.VectorSubcoreMesh(core_axis_name="c", subcore_axis_name="s")


_SC_PARAMS = pltpu.CompilerParams(use_tc_tiling_on_sc=False)


def _worker_span(n_groups):
    base_cnt = n_groups // _NW
    rem = n_groups - base_cnt * _NW

    def span(wid):
        n_mine = base_cnt + (wid < rem).astype(jnp.int32)
        row0 = wid * base_cnt + jnp.minimum(wid, rem)
        return row0, n_mine

    return base_cnt, rem, span


def _load_idx(src2, idx_v, row0, n_mine, base_cnt):
    pltpu.sync_copy(src2.at[pl.ds(row0, base_cnt)],
                    idx_v.at[pl.ds(0, base_cnt)])

    @pl.when(n_mine > base_cnt)
    def _():
        pltpu.sync_copy(src2.at[pl.ds(row0 + base_cnt, 1)],
                        idx_v.at[pl.ds(base_cnt, 1)])


def _sc_gather_sum(ta, tb, src2, dst2):
    n, d = ta.shape
    n_groups = src2.shape[0]
    e = n_groups * _G
    base_cnt, rem, span = _worker_span(n_groups)
    max_cnt = base_cnt + (1 if rem else 0)
    n_outer = (max_cnt + 1) // 2 + 1
    lanes = d // 16

    @functools.partial(
        pl.kernel, mesh=_sc_mesh(),
        out_type=jax.ShapeDtypeStruct((e, d), F32),
        compiler_params=_SC_PARAMS,
        scratch_types=[
            pltpu.VMEM((max_cnt, _G), jnp.int32),
            pltpu.VMEM((max_cnt, _G), jnp.int32),
            pltpu.VMEM((_G, d), F32), pltpu.VMEM((_G, d), F32),
            pltpu.VMEM((_G, d), F32), pltpu.VMEM((_G, d), F32),
        ] + [pltpu.SemaphoreType.DMA] * 6,
    )
    def k(ta_hbm, tb_hbm, src_hbm, dst_hbm, h1_hbm,
          idx_s, idx_d, rs0, rs1, rd0, rd1, *sems):
        rows_s, rows_d = (rs0, rs1), (rd0, rd1)
        sem_gs, sem_gd = sems[0:2], sems[2:4]
        sem_o = sems[4:6]
        wid = lax.axis_index("s") * 2 + lax.axis_index("c")
        row0, n_mine = span(wid)
        _load_idx(src_hbm, idx_s, row0, n_mine, base_cnt)
        _load_idx(dst_hbm, idx_d, row0, n_mine, base_cnt)

        def body(i, carry):
            for b in (0, 1):
                g = i * 2 + b
                ok = g < n_mine

                @pl.when(jnp.logical_and(ok, g >= 2))
                def _drain_writes():
                    base = (row0 + g) * _G
                    pltpu.make_async_copy(
                        rows_s[b], h1_hbm.at[pl.ds(base, _G)],
                        sem_o[b]).wait()

                @pl.when(ok)
                def _issue_gathers():
                    pltpu.async_copy(ta_hbm.at[idx_s.at[g]], rows_s[b],
                                     sem_gs[b])
                    pltpu.async_copy(tb_hbm.at[idx_d.at[g]], rows_d[b],
                                     sem_gd[b])

                gp = g - 1
                bp = 1 - b

                @pl.when(jnp.logical_and(gp >= 0, gp < n_mine))
                def _complete_prev():
                    pltpu.make_async_copy(ta_hbm.at[idx_s.at[gp]],
                                          rows_s[bp], sem_gs[bp]).wait()
                    pltpu.make_async_copy(tb_hbm.at[idx_d.at[gp]],
                                          rows_d[bp], sem_gd[bp]).wait()

                    def add_row(r, c):
                        for j in range(lanes):
                            sl = pl.ds(j * 16, 16)
                            rows_s[bp][r, sl] = (rows_s[bp][r, sl]
                                                 + rows_d[bp][r, sl])
                        return c

                    lax.fori_loop(0, _G, add_row, 0)
                    base = (row0 + gp) * _G
                    pltpu.async_copy(rows_s[bp], h1_hbm.at[pl.ds(base, _G)],
                                     sem_o[bp])
            return carry

        lax.fori_loop(0, n_outer, body, 0)
        for b in (0, 1):
            base = row0 * _G
            pltpu.make_async_copy(rows_s[b], h1_hbm.at[pl.ds(base, _G)],
                                  sem_o[b]).wait()

    return k(ta, tb, src2, dst2)


def _sc_scatter(vals, dst2, n):
    e, d = vals.shape
    n_groups = dst2.shape[0]
    zeros = jnp.zeros((n, d), F32)
    base_cnt, rem, span = _worker_span(n_groups)
    max_cnt = base_cnt + (1 if rem else 0)
    n_outer = (max_cnt + 1) // 2 + 1

    @functools.partial(
        pl.kernel, mesh=_sc_mesh(),
        out_type=jax.ShapeDtypeStruct((2, n, d), F32),
        compiler_params=_SC_PARAMS,
        scratch_types=[
            pltpu.VMEM((max_cnt, _G), jnp.int32),
            pltpu.VMEM((_G, d), F32), pltpu.VMEM((_G, d), F32),
            pltpu.VMEM_SHARED((n, d), F32),
        ] + [pltpu.SemaphoreType.DMA] * 4,
    )
    def k(vals_hbm, dst_hbm, zeros_hbm, out_hbm, idx_d, v0, v1, acc, *sems):
        val_v = (v0, v1)
        sem_v, sem_sc = sems[0:2], sems[2:4]
        cid = lax.axis_index("c")
        sid = lax.axis_index("s")
        wid = sid * 2 + cid
        row0, n_mine = span(wid)

        @pl.when(sid == 0)
        def _():
            pltpu.sync_copy(zeros_hbm, acc)

        _load_idx(dst_hbm, idx_d, row0, n_mine, base_cnt)
        plsc.subcore_barrier()

        def body(i, carry):
            for b in (0, 1):
                g = i * 2 + b
                ok = g < n_mine

                @pl.when(jnp.logical_and(ok, g >= 2))
                def _drain_scatter():
                    pltpu.make_async_copy(val_v[b], acc.at[idx_d.at[g]],
                                          sem_sc[b]).wait()

                @pl.when(ok)
                def _issue_load():
                    base = (row0 + g) * _G
                    pltpu.async_copy(vals_hbm.at[pl.ds(base, _G)], val_v[b],
                                     sem_v[b])

                gp = g - 1
                bp = 1 - b

                @pl.when(jnp.logical_and(gp >= 0, gp < n_mine))
                def _scatter_prev():
                    base = (row0 + gp) * _G
                    pltpu.make_async_copy(vals_hbm.at[pl.ds(base, _G)],
                                          val_v[bp], sem_v[bp]).wait()
                    pltpu.async_copy(val_v[bp], acc.at[idx_d.at[gp]],
                                     sem_sc[bp], add=True)
            return carry

        lax.fori_loop(0, n_outer, body, 0)
        for b in (0, 1):
            pltpu.make_async_copy(val_v[b], acc.at[idx_d.at[0]],
                                  sem_sc[b]).wait()
        plsc.subcore_barrier()
        rows = n // 16
        pltpu.sync_copy(acc.at[pl.ds(sid * rows, rows)],
                        out_hbm.at[cid].at[pl.ds(sid * rows, rows)])

    return k(vals, dst2, zeros)


def _sc_count(dst2, n, d):
    n_groups = dst2.shape[0]
    zeros = jnp.zeros((n, d), F32)
    ones = jnp.ones((_G, d), F32)
    base_cnt, rem, span = _worker_span(n_groups)
    max_cnt = base_cnt + (1 if rem else 0)
    n_outer = (max_cnt + 1) // 2 + 1

    @functools.partial(
        pl.kernel, mesh=_sc_mesh(),
        out_type=jax.ShapeDtypeStruct((2, n, d), F32),
        compiler_params=_SC_PARAMS,
        scratch_types=[
            pltpu.VMEM((max_cnt, _G), jnp.int32),
            pltpu.VMEM((_G, d), F32),
            pltpu.VMEM_SHARED((n, d), F32),
        ] + [pltpu.SemaphoreType.DMA] * 2,
    )
    def k(dst_hbm, zeros_hbm, ones_hbm, out_hbm, idx_d, one_v, acc, *sems):
        cid = lax.axis_index("c")
        sid = lax.axis_index("s")
        wid = sid * 2 + cid
        row0, n_mine = span(wid)

        @pl.when(sid == 0)
        def _():
            pltpu.sync_copy(zeros_hbm, acc)

        cp = pltpu.async_copy(ones_hbm, one_v, sems[0])
        _load_idx(dst_hbm, idx_d, row0, n_mine, base_cnt)
        cp.wait()
        plsc.subcore_barrier()

        def body(i, carry):
            for b in (0, 1):
                g = i * 2 + b
                ok = g < n_mine

                @pl.when(jnp.logical_and(ok, g >= 2))
                def _drain():
                    pltpu.make_async_copy(one_v, acc.at[idx_d.at[g]],
                                          sems[b]).wait()

                @pl.when(ok)
                def _issue():
                    pltpu.async_copy(one_v, acc.at[idx_d.at[g]],
                                     sems[b], add=True)
            return carry

        lax.fori_loop(0, n_outer, body, 0)
        for b in (0, 1):
            pltpu.make_async_copy(one_v, acc.at[idx_d.at[0]],
                                  sems[b]).wait()
        plsc.subcore_barrier()
        rows = n // 16
        pltpu.sync_copy(acc.at[pl.ds(sid * rows, rows)],
                        out_hbm.at[cid].at[pl.ds(sid * rows, rows)])

    return k(dst2, zeros, ones)


def kernel(edge_index, edge_feat, node_feat, graph_attr, params):
    p = params
    n_nodes = node_feat.shape[0]
    n_edges = edge_feat.shape[0]
    src2 = edge_index[0].reshape(n_edges // _G, _G)
    dst2 = edge_index[1].reshape(n_edges // _G, _G)

    ef = _mlp2(edge_feat, p['edge_enc'], tile=5000)
    nf = _mlp2(node_feat, p['node_enc'], tile=2000)
    u = _mlp2(graph_attr, p['attr_enc'], tile=1)

    d = ef.shape[1]
    cnt = _sc_count(dst2, n_nodes, d)
    ca, cb = cnt[0], cnt[1]

    for blk in p['blocks']:
        ef_in, nf_in, u_in = ef, nf, u
        if blk['dense'] is not None:
            nf_cur = _mlp2(nf, blk['dense']['node'], tile=2000)
            u_cur = _mlp2(u, blk['dense']['attr'], tile=1)
            dense_edge = blk['dense']['edge']
        else:
            nf_cur, u_cur, dense_edge = nf, u, None
        ta, tb = _ab_tables(nf_cur, blk['conv']['edge'][0][0])
        h1pre = _sc_gather_sum(ta, tb, src2, dst2)
        ef2, ef = _edge_conv(ef_in, h1pre, u_cur, dense_edge,
                             blk['conv']['edge'], tile=5000)
        ps = _sc_scatter(ef2, dst2, n_nodes)
        nf2, nf = _node_conv(nf_cur, nf_in, ps[0], ps[1], ca, cb, u_cur,
                             blk['conv']['node'], tile=2000)
        u = _attr_conv(ps[0], ps[1], nf2, u_cur, u_in,
                       blk['conv']['attr'], n_edges)

    nq = _set2set(nf, p['node_s2s'], tile=10000)
    eq = _set2set(ef, p['edge_s2s'], tile=10000)
    return _out_head(nq, eq, u, p['out'])


# speedup vs baseline: 2.5789x; 1.0441x over previous
import functools

import jax
import jax.numpy as jnp
from jax import lax
from jax.experimental import pallas as pl
from jax.experimental.pallas import tpu as pltpu
from jax.experimental.pallas import tpu_sc as plsc

F32 = jnp.float32
_G = 128
_NW = 32


def _sp(x):
    return jnp.maximum(x, 0.0) + jnp.log(1.0 + jnp.exp(-jnp.abs(x)))


def _sig(x):
    return 1.0 / (1.0 + jnp.exp(-x))


def _dot(a, b):
    return lax.dot_general(a, b, (((1,), (0,)), ((), ())),
                           preferred_element_type=F32)


def _mlp2(x, layers, tile):
    (w1, b1), (w2, b2) = layers
    r, din = x.shape
    dh, dout = w1.shape[1], w2.shape[1]
    grid = r // tile

    def body(x_ref, w1_ref, b1_ref, w2_ref, b2_ref, o_ref):
        h = _sp(_dot(x_ref[...], w1_ref[...]) + b1_ref[...])
        o_ref[...] = _sp(_dot(h, w2_ref[...]) + b2_ref[...])

    return pl.pallas_call(
        body,
        grid=(grid,),
        in_specs=[
            pl.BlockSpec((tile, din), lambda i: (i, 0)),
            pl.BlockSpec((din, dh), lambda i: (0, 0)),
            pl.BlockSpec((1, dh), lambda i: (0, 0)),
            pl.BlockSpec((dh, dout), lambda i: (0, 0)),
            pl.BlockSpec((1, dout), lambda i: (0, 0)),
        ],
        out_specs=pl.BlockSpec((tile, dout), lambda i: (i, 0)),
        out_shape=jax.ShapeDtypeStruct((r, dout), F32),
    )(x, w1, b1.reshape(1, -1), w2, b2.reshape(1, -1))


def _ab_tables(nf_cur, w1):
    d = nf_cur.shape[1]
    ws, wd = w1[:d], w1[d:2 * d]
    n = nf_cur.shape[0]
    tile = 2000
    dh = w1.shape[1]

    def body(nf_ref, ws_ref, wd_ref, a_ref, b_ref):
        x = nf_ref[...]
        a_ref[...] = _dot(x, ws_ref[...])
        b_ref[...] = _dot(x, wd_ref[...])

    return pl.pallas_call(
        body,
        grid=(n // tile,),
        in_specs=[pl.BlockSpec((tile, d), lambda i: (i, 0)),
                  pl.BlockSpec((d, dh), lambda i: (0, 0)),
                  pl.BlockSpec((d, dh), lambda i: (0, 0))],
        out_specs=[pl.BlockSpec((tile, dh), lambda i: (i, 0))] * 2,
        out_shape=[jax.ShapeDtypeStruct((n, dh), F32)] * 2,
    )(nf_cur, ws, wd)


def _edge_conv(ef_in, h1pre, u_cur, dense_layers, conv_layers, tile):
    (w1, b1), (w2, b2), (w3, b3) = conv_layers
    d = ef_in.shape[1]
    we, wu = w1[2 * d:3 * d], w1[3 * d:]
    dh = w1.shape[1]
    e = ef_in.shape[0]
    grid = e // tile
    has_dense = dense_layers is not None
    weights = [we, wu, b1.reshape(1, -1), w2, b2.reshape(1, -1),
               w3, b3.reshape(1, -1)]
    if has_dense:
        (dw1, db1), (dw2, db2) = dense_layers
        weights += [dw1, db1.reshape(1, -1), dw2, db2.reshape(1, -1)]

    def body(ef_ref, h1_ref, u_ref, *refs):
        wrefs = refs[:len(weights)]
        e2_ref, res_ref = refs[len(weights):]
        we_r, wu_r, b1_r, w2_r, b2_r, w3_r, b3_r = wrefs[:7]
        x = ef_ref[...]
        xin = x
        if has_dense:
            dw1_r, db1_r, dw2_r, db2_r = wrefs[7:]
            x = _sp(_dot(x, dw1_r[...]) + db1_r[...])
            x = _sp(_dot(x, dw2_r[...]) + db2_r[...])
        h = (h1_ref[...] + _dot(x, we_r[...])
             + _dot(u_ref[...], wu_r[...]) + b1_r[...])
        h = _sp(h)
        h = _sp(_dot(h, w2_r[...]) + b2_r[...])
        e2 = _sp(_dot(h, w3_r[...]) + b3_r[...])
        e2_ref[...] = e2
        res_ref[...] = e2 + xin

    w_specs = [pl.BlockSpec(w.shape, lambda i: (0, 0)) for w in weights]
    dout = w3.shape[1]
    return pl.pallas_call(
        body,
        grid=(grid,),
        in_specs=[pl.BlockSpec((tile, d), lambda i: (i, 0)),
                  pl.BlockSpec((tile, dh), lambda i: (i, 0)),
                  pl.BlockSpec((1, d), lambda i: (0, 0))] + w_specs,
        out_specs=[pl.BlockSpec((tile, dout), lambda i: (i, 0))] * 2,
        out_shape=[jax.ShapeDtypeStruct((e, dout), F32)] * 2,
    )(ef_in, h1pre, u_cur, *weights)


def _node_conv(nf_cur, nf_in, pa, pb, ca, cb, u_cur, conv_layers, tile):
    (w1, b1), (w2, b2), (w3, b3) = conv_layers
    d = nf_cur.shape[1]
    wn, we, wu = w1[:d], w1[d:2 * d], w1[2 * d:]
    n = nf_cur.shape[0]
    grid = n // tile
    weights = [wn, we, wu, b1.reshape(1, -1), w2, b2.reshape(1, -1),
               w3, b3.reshape(1, -1)]

    def body(nc_ref, ni_ref, pa_ref, pb_ref, ca_ref, cb_ref, u_ref, *refs):
        wn_r, we_r, wu_r, b1_r, w2_r, b2_r, w3_r, b3_r = refs[:8]
        n2_ref, res_ref = refs[8:]
        emean = (pa_ref[...] + pb_ref[...]) / jnp.maximum(
            ca_ref[...] + cb_ref[...], 1.0)
        h = (_dot(nc_ref[...], wn_r[...]) + _dot(emean, we_r[...])
             + _dot(u_ref[...], wu_r[...]) + b1_r[...])
        h = _sp(h)
        h = _sp(_dot(h, w2_r[...]) + b2_r[...])
        n2 = _sp(_dot(h, w3_r[...]) + b3_r[...])
        n2_ref[...] = n2
        res_ref[...] = n2 + ni_ref[...]

    row_spec = pl.BlockSpec((tile, d), lambda i: (i, 0))
    w_specs = [pl.BlockSpec(w.shape, lambda i: (0, 0)) for w in weights]
    dout = w3.shape[1]
    return pl.pallas_call(
        body,
        grid=(grid,),
        in_specs=[row_spec] * 6 + [pl.BlockSpec((1, d), lambda i: (0, 0))]
        + w_specs,
        out_specs=[pl.BlockSpec((tile, dout), lambda i: (i, 0))] * 2,
        out_shape=[jax.ShapeDtypeStruct((n, dout), F32)] * 2,
    )(nf_cur, nf_in, pa, pb, ca, cb, u_cur, *weights)


def _attr_conv(pa, pb, nf2, u_cur, u_in, conv_layers, n_edges):
    (w1, b1), (w2, b2), (w3, b3) = conv_layers
    d = u_cur.shape[1]
    we, wn, wu = w1[:d], w1[d:2 * d], w1[2 * d:]
    n = nf2.shape[0]
    weights = [we, wn, wu, b1.reshape(1, -1), w2, b2.reshape(1, -1),
               w3, b3.reshape(1, -1)]

    def body(pa_ref, pb_ref, nf2_ref, uc_ref, ui_ref, *refs):
        we_r, wn_r, wu_r, b1_r, w2_r, b2_r, w3_r, b3_r = refs[:8]
        o_ref = refs[8]
        mean_ef = jnp.sum(pa_ref[...] + pb_ref[...], axis=0,
                          keepdims=True) * (1.0 / n_edges)
        mean_nf = jnp.sum(nf2_ref[...], axis=0, keepdims=True) * (1.0 / n)
        h = (_dot(mean_ef, we_r[...]) + _dot(mean_nf, wn_r[...])
             + _dot(uc_ref[...], wu_r[...]) + b1_r[...])
        h = _sp(h)
        h = _sp(_dot(h, w2_r[...]) + b2_r[...])
        o_ref[...] = _sp(_dot(h, w3_r[...]) + b3_r[...]) + ui_ref[...]

    return pl.pallas_call(
        body,
        out_shape=jax.ShapeDtypeStruct((1, w3.shape[1]), F32),
    )(pa, pb, nf2, u_cur, u_in, *weights)


def _set2set(feat, p, tile):
    wi, wh, b = p['Wi'], p['Wh'], p['b']
    r, d = feat.shape
    ntiles = r // tile

    def body(f_ref, wi_ref, wh_ref, b_ref, q_ref,
             h_ref, c_ref, qs_ref, r_ref, m_ref, s_ref):
        it = pl.program_id(0)
        t = pl.program_id(1)

        @pl.when(t == 0)
        def _start_iter():
            @pl.when(it == 0)
            def _init():
                h_ref[...] = jnp.zeros((1, d), F32)
                c_ref[...] = jnp.zeros((1, d), F32)
                qs_ref[...] = jnp.zeros((1, 2 * d), F32)

            gates = (_dot(qs_ref[...], wi_ref[...])
                     + _dot(h_ref[...], wh_ref[...]) + b_ref[...])
            gi = _sig(gates[:, :d])
            gf = _sig(gates[:, d:2 * d])
            gg = jnp.tanh(gates[:, 2 * d:3 * d])
            go = _sig(gates[:, 3 * d:])
            c = gf * c_ref[...] + gi * gg
            c_ref[...] = c
            h_ref[...] = go * jnp.tanh(c)
            r_ref[...] = jnp.zeros((1, d), F32)
            m_ref[0, 0] = -1e30
            s_ref[0, 0] = 0.0

        f = f_ref[...]
        h = h_ref[...]
        logits = lax.dot_general(f, h, (((1,), (1,)), ((), ())),
                                 preferred_element_type=F32)
        m_old = m_ref[0, 0]
        m_new = jnp.maximum(m_old, jnp.max(logits))
        corr = jnp.exp(m_old - m_new)
        ex = jnp.exp(logits - m_new)
        s_ref[0, 0] = s_ref[0, 0] * corr + jnp.sum(ex)
        r_ref[...] = r_ref[...] * corr + lax.dot_general(
            ex, f, (((0,), (0,)), ((), ())), preferred_element_type=F32)
        m_ref[0, 0] = m_new

        @pl.when(t == ntiles - 1)
        def _end_iter():
            rvec = r_ref[...] / s_ref[0, 0]
            q = jnp.concatenate([h_ref[...], rvec], axis=1)
            qs_ref[...] = q

            @pl.when(it == 2)
            def _emit():
                q_ref[...] = q

    return pl.pallas_call(
        body,
        grid=(3, ntiles),
        in_specs=[
            pl.BlockSpec((tile, d), lambda it, t: (t, 0)),
            pl.BlockSpec(wi.shape, lambda it, t: (0, 0)),
            pl.BlockSpec(wh.shape, lambda it, t: (0, 0)),
            pl.BlockSpec((1, 4 * d), lambda it, t: (0, 0)),
        ],
        out_specs=pl.BlockSpec((1, 2 * d), lambda it, t: (0, 0)),
        out_shape=jax.ShapeDtypeStruct((1, 2 * d), F32),
        scratch_shapes=[
            pltpu.VMEM((1, d), F32), pltpu.VMEM((1, d), F32),
            pltpu.VMEM((1, 2 * d), F32), pltpu.VMEM((1, d), F32),
            pltpu.SMEM((1, 1), F32), pltpu.SMEM((1, 1), F32),
        ],
    )(feat, wi, wh, b.reshape(1, -1))


def _out_head(nq, eq, u, layers):
    (w1, b1), (w2, b2), (w3, b3) = layers
    dq = nq.shape[1]
    wn, we, wu = w1[:dq], w1[dq:2 * dq], w1[2 * dq:]
    weights = [wn, we, wu, b1.reshape(1, -1), w2, b2.reshape(1, -1),
               w3, b3.reshape(1, -1)]

    def body(nq_ref, eq_ref, u_ref, *refs):
        wn_r, we_r, wu_r, b1_r, w2_r, b2_r, w3_r, b3_r = refs[:8]
        o_ref = refs[8]
        h = (_dot(nq_ref[...], wn_r[...]) + _dot(eq_ref[...], we_r[...])
             + _dot(u_ref[...], wu_r[...]) + b1_r[...])
        h = _sp(h)
        h = _sp(_dot(h, w2_r[...]) + b2_r[...])
        o_ref[...] = _sig(_dot(h, w3_r[...]) + b3_r[...])

    return pl.pallas_call(
        body,
        out_shape=jax.ShapeDtypeStruct((1, 1), F32),
    )(nq, eq, u, *weights)


def _sc_mesh():
    return plsc.VectorSubcoreMesh(core_axis_name="c", subcore_axis_name="s")


_SC_PARAMS = pltpu.CompilerParams(use_tc_tiling_on_sc=False)


def _worker_span(n_groups):
    base_cnt = n_groups // _NW
    rem = n_groups - base_cnt * _NW

    def span(wid):
        n_mine = base_cnt + (wid < rem).astype(jnp.int32)
        row0 = wid * base_cnt + jnp.minimum(wid, rem)
        return row0, n_mine

    return base_cnt, rem, span


def _load_idx(src2, idx_v, row0, n_mine, base_cnt):
    pltpu.sync_copy(src2.at[pl.ds(row0, base_cnt)],
                    idx_v.at[pl.ds(0, base_cnt)])

    @pl.when(n_mine > base_cnt)
    def _():
        pltpu.sync_copy(src2.at[pl.ds(row0 + base_cnt, 1)],
                        idx_v.at[pl.ds(base_cnt, 1)])


def _sc_gather_sum(ta, tb, src2, dst2):
    n, d = ta.shape
    n_groups = src2.shape[0]
    e = n_groups * _G
    base_cnt, rem, span = _worker_span(n_groups)
    max_cnt = base_cnt + (1 if rem else 0)
    n_outer = (max_cnt + 1) // 2 + 1
    lanes = d // 16

    @functools.partial(
        pl.kernel, mesh=_sc_mesh(),
        out_type=jax.ShapeDtypeStruct((e, d), F32),
        compiler_params=_SC_PARAMS,
        scratch_types=[
            pltpu.VMEM((max_cnt, _G), jnp.int32),
            pltpu.VMEM((max_cnt, _G), jnp.int32),
            pltpu.VMEM((_G, d), F32), pltpu.VMEM((_G, d), F32),
            pltpu.VMEM((_G, d), F32), pltpu.VMEM((_G, d), F32),
        ] + [pltpu.SemaphoreType.DMA] * 6,
    )
    def k(ta_hbm, tb_hbm, src_hbm, dst_hbm, h1_hbm,
          idx_s, idx_d, rs0, rs1, rd0, rd1, *sems):
        rows_s, rows_d = (rs0, rs1), (rd0, rd1)
        sem_gs, sem_gd = sems[0:2], sems[2:4]
        sem_o = sems[4:6]
        wid = lax.axis_index("s") * 2 + lax.axis_index("c")
        row0, n_mine = span(wid)
        _load_idx(src_hbm, idx_s, row0, n_mine, base_cnt)
        _load_idx(dst_hbm, idx_d, row0, n_mine, base_cnt)

        def body(i, carry):
            for b in (0, 1):
                g = i * 2 + b
                ok = g < n_mine

                @pl.when(jnp.logical_and(ok, g >= 2))
                def _drain_writes():
                    base = (row0 + g) * _G
                    pltpu.make_async_copy(
                        rows_s[b], h1_hbm.at[pl.ds(base, _G)],
                        sem_o[b]).wait()

                @pl.when(ok)
                def _issue_gathers():
                    pltpu.async_copy(ta_hbm.at[idx_s.at[g]], rows_s[b],
                                     sem_gs[b])
                    pltpu.async_copy(tb_hbm.at[idx_d.at[g]], rows_d[b],
                                     sem_gd[b])

                gp = g - 1
                bp = 1 - b

                @pl.when(jnp.logical_and(gp >= 0, gp < n_mine))
                def _complete_prev():
                    pltpu.make_async_copy(ta_hbm.at[idx_s.at[gp]],
                                          rows_s[bp], sem_gs[bp]).wait()
                    pltpu.make_async_copy(tb_hbm.at[idx_d.at[gp]],
                                          rows_d[bp], sem_gd[bp]).wait()

                    def add_row(r, c):
                        for j in range(lanes):
                            sl = pl.ds(j * 16, 16)
                            rows_s[bp][r, sl] = (rows_s[bp][r, sl]
                                                 + rows_d[bp][r, sl])
                        return c

                    lax.fori_loop(0, _G, add_row, 0)
                    base = (row0 + gp) * _G
                    pltpu.async_copy(rows_s[bp], h1_hbm.at[pl.ds(base, _G)],
                                     sem_o[bp])
            return carry

        lax.fori_loop(0, n_outer, body, 0)
        for b in (0, 1):
            base = row0 * _G
            pltpu.make_async_copy(rows_s[b], h1_hbm.at[pl.ds(base, _G)],
                                  sem_o[b]).wait()

    return k(ta, tb, src2, dst2)


def _sc_scatter(vals, dst2, n):
    e, d = vals.shape
    n_groups = dst2.shape[0]
    zeros = jnp.zeros((n, d), F32)
    base_cnt, rem, span = _worker_span(n_groups)
    max_cnt = base_cnt + (1 if rem else 0)
    n_outer = (max_cnt + 1) // 2 + 1

    @functools.partial(
        pl.kernel, mesh=_sc_mesh(),
        out_type=jax.ShapeDtypeStruct((2, n, d), F32),
        compiler_params=_SC_PARAMS,
        scratch_types=[
            pltpu.VMEM((max_cnt, _G), jnp.int32),
            pltpu.VMEM((_G, d), F32), pltpu.VMEM((_G, d), F32),
            pltpu.VMEM_SHARED((n, d), F32),
        ] + [pltpu.SemaphoreType.DMA] * 4,
    )
    def k(vals_hbm, dst_hbm, zeros_hbm, out_hbm, idx_d, v0, v1, acc, *sems):
        val_v = (v0, v1)
        sem_v, sem_sc = sems[0:2], sems[2:4]
        cid = lax.axis_index("c")
        sid = lax.axis_index("s")
        wid = sid * 2 + cid
        row0, n_mine = span(wid)

        @pl.when(sid == 0)
        def _():
            pltpu.sync_copy(zeros_hbm, acc)

        _load_idx(dst_hbm, idx_d, row0, n_mine, base_cnt)
        plsc.subcore_barrier()

        def body(i, carry):
            for b in (0, 1):
                g = i * 2 + b
                ok = g < n_mine

                @pl.when(jnp.logical_and(ok, g >= 2))
                def _drain_scatter():
                    pltpu.make_async_copy(val_v[b], acc.at[idx_d.at[g]],
                                          sem_sc[b]).wait()

                @pl.when(ok)
                def _issue_load():
                    base = (row0 + g) * _G
                    pltpu.async_copy(vals_hbm.at[pl.ds(base, _G)], val_v[b],
                                     sem_v[b])

                gp = g - 1
                bp = 1 - b

                @pl.when(jnp.logical_and(gp >= 0, gp < n_mine))
                def _scatter_prev():
                    base = (row0 + gp) * _G
                    pltpu.make_async_copy(vals_hbm.at[pl.ds(base, _G)],
                                          val_v[bp], sem_v[bp]).wait()
                    pltpu.async_copy(val_v[bp], acc.at[idx_d.at[gp]],
                                     sem_sc[bp], add=True)
            return carry

        lax.fori_loop(0, n_outer, body, 0)
        for b in (0, 1):
            pltpu.make_async_copy(val_v[b], acc.at[idx_d.at[0]],
                                  sem_sc[b]).wait()
        plsc.subcore_barrier()
        rows = n // 16
        pltpu.sync_copy(acc.at[pl.ds(sid * rows, rows)],
                        out_hbm.at[cid].at[pl.ds(sid * rows, rows)])

    return k(vals, dst2, zeros)


def _sc_count(dst2, n, d):
    n_groups = dst2.shape[0]
    zeros = jnp.zeros((n, d), F32)
    ones = jnp.ones((_G, d), F32)
    base_cnt, rem, span = _worker_span(n_groups)
    max_cnt = base_cnt + (1 if rem else 0)
    n_outer = (max_cnt + 1) // 2 + 1

    @functools.partial(
        pl.kernel, mesh=_sc_mesh(),
        out_type=jax.ShapeDtypeStruct((2, n, d), F32),
        compiler_params=_SC_PARAMS,
        scratch_types=[
            pltpu.VMEM((max_cnt, _G), jnp.int32),
            pltpu.VMEM((_G, d), F32),
            pltpu.VMEM_SHARED((n, d), F32),
        ] + [pltpu.SemaphoreType.DMA] * 2,
    )
    def k(dst_hbm, zeros_hbm, ones_hbm, out_hbm, idx_d, one_v, acc, *sems):
        cid = lax.axis_index("c")
        sid = lax.axis_index("s")
        wid = sid * 2 + cid
        row0, n_mine = span(wid)

        @pl.when(sid == 0)
        def _():
            pltpu.sync_copy(zeros_hbm, acc)

        cp = pltpu.async_copy(ones_hbm, one_v, sems[0])
        _load_idx(dst_hbm, idx_d, row0, n_mine, base_cnt)
        cp.wait()
        plsc.subcore_barrier()

        def body(i, carry):
            for b in (0, 1):
                g = i * 2 + b
                ok = g < n_mine

                @pl.when(jnp.logical_and(ok, g >= 2))
                def _drain():
                    pltpu.make_async_copy(one_v, acc.at[idx_d.at[g]],
                                          sems[b]).wait()

                @pl.when(ok)
                def _issue():
                    pltpu.async_copy(one_v, acc.at[idx_d.at[g]],
                                     sems[b], add=True)
            return carry

        lax.fori_loop(0, n_outer, body, 0)
        for b in (0, 1):
            pltpu.make_async_copy(one_v, acc.at[idx_d.at[0]],
                                  sems[b]).wait()
        plsc.subcore_barrier()
        rows = n // 16
        pltpu.sync_copy(acc.at[pl.ds(sid * rows, rows)],
                        out_hbm.at[cid].at[pl.ds(sid * rows, rows)])

    return k(dst2, zeros, ones)


def kernel(edge_index, edge_feat, node_feat, graph_attr, params):
    p = params
    n_nodes = node_feat.shape[0]
    n_edges = edge_feat.shape[0]
    src2 = edge_index[0].reshape(n_edges // _G, _G)
    dst2 = edge_index[1].reshape(n_edges // _G, _G)

    ef = _mlp2(edge_feat, p['edge_enc'], tile=10000)
    nf = _mlp2(node_feat, p['node_enc'], tile=2000)
    u = _mlp2(graph_attr, p['attr_enc'], tile=1)

    d = ef.shape[1]
    cnt = _sc_count(dst2, n_nodes, d)
    ca, cb = cnt[0], cnt[1]

    for blk in p['blocks']:
        ef_in, nf_in, u_in = ef, nf, u
        if blk['dense'] is not None:
            nf_cur = _mlp2(nf, blk['dense']['node'], tile=2000)
            u_cur = _mlp2(u, blk['dense']['attr'], tile=1)
            dense_edge = blk['dense']['edge']
        else:
            nf_cur, u_cur, dense_edge = nf, u, None
        ta, tb = _ab_tables(nf_cur, blk['conv']['edge'][0][0])
        h1pre = _sc_gather_sum(ta, tb, src2, dst2)
        ef2, ef = _edge_conv(ef_in, h1pre, u_cur, dense_edge,
                             blk['conv']['edge'], tile=8000)
        ps = _sc_scatter(ef2, dst2, n_nodes)
        nf2, nf = _node_conv(nf_cur, nf_in, ps[0], ps[1], ca, cb, u_cur,
                             blk['conv']['node'], tile=2000)
        u = _attr_conv(ps[0], ps[1], nf2, u_cur, u_in,
                       blk['conv']['attr'], n_edges)

    nq = _set2set(nf, p['node_s2s'], tile=10000)
    eq = _set2set(ef, p['edge_s2s'], tile=20000)
    return _out_head(nq, eq, u, p['out'])


# speedup vs baseline: 2.6336x; 1.0212x over previous
import functools

import jax
import jax.numpy as jnp
from jax import lax
from jax.experimental import pallas as pl
from jax.experimental.pallas import tpu as pltpu
from jax.experimental.pallas import tpu_sc as plsc

F32 = jnp.float32
_G = 128
_NW = 32


def _sp(x):
    return jnp.maximum(x, 0.0) + jnp.log(1.0 + jnp.exp(-jnp.abs(x)))


def _sig(x):
    return 1.0 / (1.0 + jnp.exp(-x))


def _dot(a, b):
    return lax.dot_general(a, b, (((1,), (0,)), ((), ())),
                           preferred_element_type=F32)


def _mlp2(x, layers, tile):
    (w1, b1), (w2, b2) = layers
    r, din = x.shape
    dh, dout = w1.shape[1], w2.shape[1]
    grid = r // tile

    def body(x_ref, w1_ref, b1_ref, w2_ref, b2_ref, o_ref):
        h = _sp(_dot(x_ref[...], w1_ref[...]) + b1_ref[...])
        o_ref[...] = _sp(_dot(h, w2_ref[...]) + b2_ref[...])

    return pl.pallas_call(
        body,
        grid=(grid,),
        in_specs=[
            pl.BlockSpec((tile, din), lambda i: (i, 0)),
            pl.BlockSpec((din, dh), lambda i: (0, 0)),
            pl.BlockSpec((1, dh), lambda i: (0, 0)),
            pl.BlockSpec((dh, dout), lambda i: (0, 0)),
            pl.BlockSpec((1, dout), lambda i: (0, 0)),
        ],
        out_specs=pl.BlockSpec((tile, dout), lambda i: (i, 0)),
        out_shape=jax.ShapeDtypeStruct((r, dout), F32),
    )(x, w1, b1.reshape(1, -1), w2, b2.reshape(1, -1))


def _mlp2_ab(x, u, layers_x, layers_u, w1conv, tile):
    (w1, b1), (w2, b2) = layers_x
    (v1, c1), (v2, c2) = layers_u
    r, din = x.shape
    dh, dout = w1.shape[1], w2.shape[1]
    ws, wd = w1conv[:dout], w1conv[dout:2 * dout]
    dab = w1conv.shape[1]

    def body(x_ref, u_ref, w1_ref, b1_ref, w2_ref, b2_ref,
             v1_ref, c1_ref, v2_ref, c2_ref, ws_ref, wd_ref,
             o_ref, uo_ref, a_ref, bt_ref):
        h = _sp(_dot(x_ref[...], w1_ref[...]) + b1_ref[...])
        o = _sp(_dot(h, w2_ref[...]) + b2_ref[...])
        o_ref[...] = o
        a_ref[...] = _dot(o, ws_ref[...])
        bt_ref[...] = _dot(o, wd_ref[...])

        @pl.when(pl.program_id(0) == 0)
        def _():
            hu = _sp(_dot(u_ref[...], v1_ref[...]) + c1_ref[...])
            uo_ref[...] = _sp(_dot(hu, v2_ref[...]) + c2_ref[...])

    wspec = lambda w: pl.BlockSpec(w.shape, lambda i: (0, 0))
    args = [x, u, w1, b1.reshape(1, -1), w2, b2.reshape(1, -1),
            v1, c1.reshape(1, -1), v2, c2.reshape(1, -1), ws, wd]
    return pl.pallas_call(
        body,
        grid=(r // tile,),
        in_specs=[pl.BlockSpec((tile, din), lambda i: (i, 0))]
        + [wspec(a) for a in args[1:]],
        out_specs=[pl.BlockSpec((tile, dout), lambda i: (i, 0)),
                   pl.BlockSpec((1, dout), lambda i: (0, 0)),
                   pl.BlockSpec((tile, dab), lambda i: (i, 0)),
                   pl.BlockSpec((tile, dab), lambda i: (i, 0))],
        out_shape=[jax.ShapeDtypeStruct((r, dout), F32),
                   jax.ShapeDtypeStruct((1, dout), F32),
                   jax.ShapeDtypeStruct((r, dab), F32),
                   jax.ShapeDtypeStruct((r, dab), F32)],
    )(*args)


def _edge_conv(ef_in, h1pre, u_cur, dense_layers, conv_layers, tile):
    (w1, b1), (w2, b2), (w3, b3) = conv_layers
    d = ef_in.shape[1]
    we, wu = w1[2 * d:3 * d], w1[3 * d:]
    dh = w1.shape[1]
    e = ef_in.shape[0]
    grid = e // tile
    has_dense = dense_layers is not None
    weights = [we, wu, b1.reshape(1, -1), w2, b2.reshape(1, -1),
               w3, b3.reshape(1, -1)]
    if has_dense:
        (dw1, db1), (dw2, db2) = dense_layers
        weights += [dw1, db1.reshape(1, -1), dw2, db2.reshape(1, -1)]

    def body(ef_ref, h1_ref, u_ref, *refs):
        wrefs = refs[:len(weights)]
        e2_ref, res_ref = refs[len(weights):]
        we_r, wu_r, b1_r, w2_r, b2_r, w3_r, b3_r = wrefs[:7]
        x = ef_ref[...]
        xin = x
        if has_dense:
            dw1_r, db1_r, dw2_r, db2_r = wrefs[7:]
            x = _sp(_dot(x, dw1_r[...]) + db1_r[...])
            x = _sp(_dot(x, dw2_r[...]) + db2_r[...])
        h = (h1_ref[...] + _dot(x, we_r[...])
             + _dot(u_ref[...], wu_r[...]) + b1_r[...])
        h = _sp(h)
        h = _sp(_dot(h, w2_r[...]) + b2_r[...])
        e2 = _sp(_dot(h, w3_r[...]) + b3_r[...])
        e2_ref[...] = e2
        res_ref[...] = e2 + xin

    w_specs = [pl.BlockSpec(w.shape, lambda i: (0, 0)) for w in weights]
    dout = w3.shape[1]
    return pl.pallas_call(
        body,
        grid=(grid,),
        in_specs=[pl.BlockSpec((tile, d), lambda i: (i, 0)),
                  pl.BlockSpec((tile, dh), lambda i: (i, 0)),
                  pl.BlockSpec((1, d), lambda i: (0, 0))] + w_specs,
        out_specs=[pl.BlockSpec((tile, dout), lambda i: (i, 0))] * 2,
        out_shape=[jax.ShapeDtypeStruct((e, dout), F32)] * 2,
    )(ef_in, h1pre, u_cur, *weights)


def _node_attr_conv(nf_cur, nf_in, pa, pb, ca, cb, u_cur, u_in,
                    conv_n, conv_a, n_edges, tile):
    (w1, b1), (w2, b2), (w3, b3) = conv_n
    (aw1, ab1), (aw2, ab2), (aw3, ab3) = conv_a
    d = nf_cur.shape[1]
    wn, we, wu = w1[:d], w1[d:2 * d], w1[2 * d:]
    awe, awn, awu = aw1[:d], aw1[d:2 * d], aw1[2 * d:]
    n = nf_cur.shape[0]
    grid = n // tile
    weights = [wn, we, wu, b1.reshape(1, -1), w2, b2.reshape(1, -1),
               w3, b3.reshape(1, -1),
               awe, awn, awu, ab1.reshape(1, -1), aw2, ab2.reshape(1, -1),
               aw3, ab3.reshape(1, -1)]

    def body(nc_ref, ni_ref, pa_ref, pb_ref, ca_ref, cb_ref,
             uc_ref, ui_ref, *refs):
        (wn_r, we_r, wu_r, b1_r, w2_r, b2_r, w3_r, b3_r,
         awe_r, awn_r, awu_r, ab1_r, aw2_r, ab2_r, aw3_r, ab3_r) = refs[:16]
        res_ref, uo_ref = refs[16:18]
        acc_e, acc_n = refs[18:]
        i = pl.program_id(0)

        @pl.when(i == 0)
        def _init():
            acc_e[...] = jnp.zeros_like(acc_e)
            acc_n[...] = jnp.zeros_like(acc_n)

        psum = pa_ref[...] + pb_ref[...]
        emean = psum / jnp.maximum(ca_ref[...] + cb_ref[...], 1.0)
        h = (_dot(nc_ref[...], wn_r[...]) + _dot(emean, we_r[...])
             + _dot(uc_ref[...], wu_r[...]) + b1_r[...])
        h = _sp(h)
        h = _sp(_dot(h, w2_r[...]) + b2_r[...])
        n2 = _sp(_dot(h, w3_r[...]) + b3_r[...])
        res_ref[...] = n2 + ni_ref[...]
        acc_e[...] = acc_e[...] + jnp.sum(psum, axis=0, keepdims=True)
        acc_n[...] = acc_n[...] + jnp.sum(n2, axis=0, keepdims=True)

        @pl.when(i == grid - 1)
        def _attr():
            mean_ef = acc_e[...] * (1.0 / n_edges)
            mean_nf = acc_n[...] * (1.0 / n)
            ha = (_dot(mean_ef, awe_r[...]) + _dot(mean_nf, awn_r[...])
                  + _dot(uc_ref[...], awu_r[...]) + ab1_r[...])
            ha = _sp(ha)
            ha = _sp(_dot(ha, aw2_r[...]) + ab2_r[...])
            uo_ref[...] = _sp(_dot(ha, aw3_r[...]) + ab3_r[...]) + ui_ref[...]

    row_spec = pl.BlockSpec((tile, d), lambda i: (i, 0))
    one_spec = pl.BlockSpec((1, d), lambda i: (0, 0))
    w_specs = [pl.BlockSpec(w.shape, lambda i: (0, 0)) for w in weights]
    dout = w3.shape[1]
    return pl.pallas_call(
        body,
        grid=(grid,),
        in_specs=[row_spec] * 6 + [one_spec, one_spec] + w_specs,
        out_specs=[pl.BlockSpec((tile, dout), lambda i: (i, 0)),
                   pl.BlockSpec((1, dout), lambda i: (0, 0))],
        out_shape=[jax.ShapeDtypeStruct((n, dout), F32),
                   jax.ShapeDtypeStruct((1, dout), F32)],
        scratch_shapes=[pltpu.VMEM((1, dout), F32),
                        pltpu.VMEM((1, dout), F32)],
    )(nf_cur, nf_in, pa, pb, ca, cb, u_cur, u_in, *weights)


def _set2set(feat, p, tile):
    wi, wh, b = p['Wi'], p['Wh'], p['b']
    r, d = feat.shape
    ntiles = r // tile

    def body(f_ref, wi_ref, wh_ref, b_ref, q_ref,
             h_ref, c_ref, qs_ref, r_ref, m_ref, s_ref):
        it = pl.program_id(0)
        t = pl.program_id(1)

        @pl.when(t == 0)
        def _start_iter():
            @pl.when(it == 0)
            def _init():
                h_ref[...] = jnp.zeros((1, d), F32)
                c_ref[...] = jnp.zeros((1, d), F32)
                qs_ref[...] = jnp.zeros((1, 2 * d), F32)

            gates = (_dot(qs_ref[...], wi_ref[...])
                     + _dot(h_ref[...], wh_ref[...]) + b_ref[...])
            gi = _sig(gates[:, :d])
            gf = _sig(gates[:, d:2 * d])
            gg = jnp.tanh(gates[:, 2 * d:3 * d])
            go = _sig(gates[:, 3 * d:])
            c = gf * c_ref[...] + gi * gg
            c_ref[...] = c
            h_ref[...] = go * jnp.tanh(c)
            r_ref[...] = jnp.zeros((1, d), F32)
            m_ref[0, 0] = -1e30
            s_ref[0, 0] = 0.0

        f = f_ref[...]
        h = h_ref[...]
        logits = lax.dot_general(f, h, (((1,), (1,)), ((), ())),
                                 preferred_element_type=F32)
        m_old = m_ref[0, 0]
        m_new = jnp.maximum(m_old, jnp.max(logits))
        corr = jnp.exp(m_old - m_new)
        ex = jnp.exp(logits - m_new)
        s_ref[0, 0] = s_ref[0, 0] * corr + jnp.sum(ex)
        r_ref[...] = r_ref[...] * corr + lax.dot_general(
            ex, f, (((0,), (0,)), ((), ())), preferred_element_type=F32)
        m_ref[0, 0] = m_new

        @pl.when(t == ntiles - 1)
        def _end_iter():
            rvec = r_ref[...] / s_ref[0, 0]
            q = jnp.concatenate([h_ref[...], rvec], axis=1)
            qs_ref[...] = q

            @pl.when(it == 2)
            def _emit():
                q_ref[...] = q

    return pl.pallas_call(
        body,
        grid=(3, ntiles),
        in_specs=[
            pl.BlockSpec((tile, d), lambda it, t: (t, 0)),
            pl.BlockSpec(wi.shape, lambda it, t: (0, 0)),
            pl.BlockSpec(wh.shape, lambda it, t: (0, 0)),
            pl.BlockSpec((1, 4 * d), lambda it, t: (0, 0)),
        ],
        out_specs=pl.BlockSpec((1, 2 * d), lambda it, t: (0, 0)),
        out_shape=jax.ShapeDtypeStruct((1, 2 * d), F32),
        scratch_shapes=[
            pltpu.VMEM((1, d), F32), pltpu.VMEM((1, d), F32),
            pltpu.VMEM((1, 2 * d), F32), pltpu.VMEM((1, d), F32),
            pltpu.SMEM((1, 1), F32), pltpu.SMEM((1, 1), F32),
        ],
    )(feat, wi, wh, b.reshape(1, -1))


def _out_head(nq, eq, u, layers):
    (w1, b1), (w2, b2), (w3, b3) = layers
    dq = nq.shape[1]
    wn, we, wu = w1[:dq], w1[dq:2 * dq], w1[2 * dq:]
    weights = [wn, we, wu, b1.reshape(1, -1), w2, b2.reshape(1, -1),
               w3, b3.reshape(1, -1)]

    def body(nq_ref, eq_ref, u_ref, *refs):
        wn_r, we_r, wu_r, b1_r, w2_r, b2_r, w3_r, b3_r = refs[:8]
        o_ref = refs[8]
        h = (_dot(nq_ref[...], wn_r[...]) + _dot(eq_ref[...], we_r[...])
             + _dot(u_ref[...], wu_r[...]) + b1_r[...])
        h = _sp(h)
        h = _sp(_dot(h, w2_r[...]) + b2_r[...])
        o_ref[...] = _sig(_dot(h, w3_r[...]) + b3_r[...])

    return pl.pallas_call(
        body,
        out_shape=jax.ShapeDtypeStruct((1, 1), F32),
    )(nq, eq, u, *weights)


def _sc_mesh():
    return plsc.VectorSubcoreMesh(core_axis_name="c", subcore_axis_name="s")


_SC_PARAMS = pltpu.CompilerParams(use_tc_tiling_on_sc=False)


def _worker_span(n_groups):
    base_cnt = n_groups // _NW
    rem = n_groups - base_cnt * _NW

    def span(wid):
        n_mine = base_cnt + (wid < rem).astype(jnp.int32)
        row0 = wid * base_cnt + jnp.minimum(wid, rem)
        return row0, n_mine

    return base_cnt, rem, span


def _load_idx(src2, idx_v, row0, n_mine, base_cnt):
    pltpu.sync_copy(src2.at[pl.ds(row0, base_cnt)],
                    idx_v.at[pl.ds(0, base_cnt)])

    @pl.when(n_mine > base_cnt)
    def _():
        pltpu.sync_copy(src2.at[pl.ds(row0 + base_cnt, 1)],
                        idx_v.at[pl.ds(base_cnt, 1)])


def _sc_gather_sum(ta, tb, src2, dst2):
    n, d = ta.shape
    n_groups = src2.shape[0]
    e = n_groups * _G
    base_cnt, rem, span = _worker_span(n_groups)
    max_cnt = base_cnt + (1 if rem else 0)
    n_outer = (max_cnt + 1) // 2 + 1
    lanes = d // 16

    @functools.partial(
        pl.kernel, mesh=_sc_mesh(),
        out_type=jax.ShapeDtypeStruct((e, d), F32),
        compiler_params=_SC_PARAMS,
        scratch_types=[
            pltpu.VMEM((max_cnt, _G), jnp.int32),
            pltpu.VMEM((max_cnt, _G), jnp.int32),
            pltpu.VMEM((_G, d), F32), pltpu.VMEM((_G, d), F32),
            pltpu.VMEM((_G, d), F32), pltpu.VMEM((_G, d), F32),
        ] + [pltpu.SemaphoreType.DMA] * 6,
    )
    def k(ta_hbm, tb_hbm, src_hbm, dst_hbm, h1_hbm,
          idx_s, idx_d, rs0, rs1, rd0, rd1, *sems):
        rows_s, rows_d = (rs0, rs1), (rd0, rd1)
        sem_gs, sem_gd = sems[0:2], sems[2:4]
        sem_o = sems[4:6]
        wid = lax.axis_index("s") * 2 + lax.axis_index("c")
        row0, n_mine = span(wid)
        _load_idx(src_hbm, idx_s, row0, n_mine, base_cnt)
        _load_idx(dst_hbm, idx_d, row0, n_mine, base_cnt)

        def body(i, carry):
            for b in (0, 1):
                g = i * 2 + b
                ok = g < n_mine

                @pl.when(jnp.logical_and(ok, g >= 2))
                def _drain_writes():
                    base = (row0 + g) * _G
                    pltpu.make_async_copy(
                        rows_s[b], h1_hbm.at[pl.ds(base, _G)],
                        sem_o[b]).wait()

                @pl.when(ok)
                def _issue_gathers():
                    pltpu.async_copy(ta_hbm.at[idx_s.at[g]], rows_s[b],
                                     sem_gs[b])
                    pltpu.async_copy(tb_hbm.at[idx_d.at[g]], rows_d[b],
                                     sem_gd[b])

                gp = g - 1
                bp = 1 - b

                @pl.when(jnp.logical_and(gp >= 0, gp < n_mine))
                def _complete_prev():
                    pltpu.make_async_copy(ta_hbm.at[idx_s.at[gp]],
                                          rows_s[bp], sem_gs[bp]).wait()
                    pltpu.make_async_copy(tb_hbm.at[idx_d.at[gp]],
                                          rows_d[bp], sem_gd[bp]).wait()

                    def add_row(r, c):
                        for j in range(lanes):
                            sl = pl.ds(j * 16, 16)
                            rows_s[bp][r, sl] = (rows_s[bp][r, sl]
                                                 + rows_d[bp][r, sl])
                        return c

                    lax.fori_loop(0, _G, add_row, 0)
                    base = (row0 + gp) * _G
                    pltpu.async_copy(rows_s[bp], h1_hbm.at[pl.ds(base, _G)],
                                     sem_o[bp])
            return carry

        lax.fori_loop(0, n_outer, body, 0)
        for b in (0, 1):
            base = row0 * _G
            pltpu.make_async_copy(rows_s[b], h1_hbm.at[pl.ds(base, _G)],
                                  sem_o[b]).wait()

    return k(ta, tb, src2, dst2)


def _sc_scatter(vals, dst2, n):
    e, d = vals.shape
    n_groups = dst2.shape[0]
    zeros = jnp.zeros((n, d), F32)
    base_cnt, rem, span = _worker_span(n_groups)
    max_cnt = base_cnt + (1 if rem else 0)
    n_outer = (max_cnt + 1) // 2 + 1

    @functools.partial(
        pl.kernel, mesh=_sc_mesh(),
        out_type=jax.ShapeDtypeStruct((2, n, d), F32),
        compiler_params=_SC_PARAMS,
        scratch_types=[
            pltpu.VMEM((max_cnt, _G), jnp.int32),
            pltpu.VMEM((_G, d), F32), pltpu.VMEM((_G, d), F32),
            pltpu.VMEM_SHARED((n, d), F32),
        ] + [pltpu.SemaphoreType.DMA] * 4,
    )
    def k(vals_hbm, dst_hbm, zeros_hbm, out_hbm, idx_d, v0, v1, acc, *sems):
        val_v = (v0, v1)
        sem_v, sem_sc = sems[0:2], sems[2:4]
        cid = lax.axis_index("c")
        sid = lax.axis_index("s")
        wid = sid * 2 + cid
        row0, n_mine = span(wid)

        @pl.when(sid == 0)
        def _():
            pltpu.sync_copy(zeros_hbm, acc)

        _load_idx(dst_hbm, idx_d, row0, n_mine, base_cnt)
        plsc.subcore_barrier()

        def body(i, carry):
            for b in (0, 1):
                g = i * 2 + b
                ok = g < n_mine

                @pl.when(jnp.logical_and(ok, g >= 2))
                def _drain_scatter():
                    pltpu.make_async_copy(val_v[b], acc.at[idx_d.at[g]],
                                          sem_sc[b]).wait()

                @pl.when(ok)
                def _issue_load():
                    base = (row0 + g) * _G
                    pltpu.async_copy(vals_hbm.at[pl.ds(base, _G)], val_v[b],
                                     sem_v[b])

                gp = g - 1
                bp = 1 - b

                @pl.when(jnp.logical_and(gp >= 0, gp < n_mine))
                def _scatter_prev():
                    base = (row0 + gp) * _G
                    pltpu.make_async_copy(vals_hbm.at[pl.ds(base, _G)],
                                          val_v[bp], sem_v[bp]).wait()
                    pltpu.async_copy(val_v[bp], acc.at[idx_d.at[gp]],
                                     sem_sc[bp], add=True)
            return carry

        lax.fori_loop(0, n_outer, body, 0)
        for b in (0, 1):
            pltpu.make_async_copy(val_v[b], acc.at[idx_d.at[0]],
                                  sem_sc[b]).wait()
        plsc.subcore_barrier()
        rows = n // 16
        pltpu.sync_copy(acc.at[pl.ds(sid * rows, rows)],
                        out_hbm.at[cid].at[pl.ds(sid * rows, rows)])

    return k(vals, dst2, zeros)


def _sc_count(dst2, n, d):
    n_groups = dst2.shape[0]
    zeros = jnp.zeros((n, d), F32)
    ones = jnp.ones((_G, d), F32)
    base_cnt, rem, span = _worker_span(n_groups)
    max_cnt = base_cnt + (1 if rem else 0)
    n_outer = (max_cnt + 1) // 2 + 1

    @functools.partial(
        pl.kernel, mesh=_sc_mesh(),
        out_type=jax.ShapeDtypeStruct((2, n, d), F32),
        compiler_params=_SC_PARAMS,
        scratch_types=[
            pltpu.VMEM((max_cnt, _G), jnp.int32),
            pltpu.VMEM((_G, d), F32),
            pltpu.VMEM_SHARED((n, d), F32),
        ] + [pltpu.SemaphoreType.DMA] * 2,
    )
    def k(dst_hbm, zeros_hbm, ones_hbm, out_hbm, idx_d, one_v, acc, *sems):
        cid = lax.axis_index("c")
        sid = lax.axis_index("s")
        wid = sid * 2 + cid
        row0, n_mine = span(wid)

        @pl.when(sid == 0)
        def _():
            pltpu.sync_copy(zeros_hbm, acc)

        cp = pltpu.async_copy(ones_hbm, one_v, sems[0])
        _load_idx(dst_hbm, idx_d, row0, n_mine, base_cnt)
        cp.wait()
        plsc.subcore_barrier()

        def body(i, carry):
            for b in (0, 1):
                g = i * 2 + b
                ok = g < n_mine

                @pl.when(jnp.logical_and(ok, g >= 2))
                def _drain():
                    pltpu.make_async_copy(one_v, acc.at[idx_d.at[g]],
                                          sems[b]).wait()

                @pl.when(ok)
                def _issue():
                    pltpu.async_copy(one_v, acc.at[idx_d.at[g]],
                                     sems[b], add=True)
            return carry

        lax.fori_loop(0, n_outer, body, 0)
        for b in (0, 1):
            pltpu.make_async_copy(one_v, acc.at[idx_d.at[0]],
                                  sems[b]).wait()
        plsc.subcore_barrier()
        rows = n // 16
        pltpu.sync_copy(acc.at[pl.ds(sid * rows, rows)],
                        out_hbm.at[cid].at[pl.ds(sid * rows, rows)])

    return k(dst2, zeros, ones)


def kernel(edge_index, edge_feat, node_feat, graph_attr, params):
    p = params
    n_nodes = node_feat.shape[0]
    n_edges = edge_feat.shape[0]
    src2 = edge_index[0].reshape(n_edges // _G, _G)
    dst2 = edge_index[1].reshape(n_edges // _G, _G)

    ef = _mlp2(edge_feat, p['edge_enc'], tile=10000)
    nf, u, ta, tb = _mlp2_ab(node_feat, graph_attr, p['node_enc'],
                             p['attr_enc'], p['blocks'][0]['conv']['edge'][0][0],
                             tile=2000)

    d = ef.shape[1]
    cnt = _sc_count(dst2, n_nodes, d)
    ca, cb = cnt[0], cnt[1]

    for blk in p['blocks']:
        ef_in, nf_in, u_in = ef, nf, u
        if blk['dense'] is not None:
            nf_cur, u_cur, ta, tb = _mlp2_ab(
                nf, u, blk['dense']['node'], blk['dense']['attr'],
                blk['conv']['edge'][0][0], tile=2000)
            dense_edge = blk['dense']['edge']
        else:
            nf_cur, u_cur, dense_edge = nf, u, None
        h1pre = _sc_gather_sum(ta, tb, src2, dst2)
        ef2, ef = _edge_conv(ef_in, h1pre, u_cur, dense_edge,
                             blk['conv']['edge'], tile=8000)
        ps = _sc_scatter(ef2, dst2, n_nodes)
        nf, u = _node_attr_conv(nf_cur, nf_in, ps[0], ps[1], ca, cb,
                                u_cur, u_in, blk['conv']['node'],
                                blk['conv']['attr'], n_edges, tile=2000)

    nq = _set2set(nf, p['node_s2s'], tile=10000)
    eq = _set2set(ef, p['edge_s2s'], tile=20000)
    return _out_head(nq, eq, u, p['out'])
